# Initial kernel scaffold; baseline (speedup 1.0000x reference)
import jax
import jax.numpy as jnp
from jax.experimental import pallas as pl


def kernel(x, xh, e, sc_W1, sc_b1, sc_W2, sc_b2, n1_W1, n1_b1, n1_W2, n1_b2, n2_W1, n2_b1, n2_W2, n2_b2, n3_W1, n3_b1, n3_W2, n3_b2):
    raise NotImplementedError("write your pallas kernel here")


# speedup vs baseline: 14.4404x; 14.4404x over previous
import functools

import jax
import jax.numpy as jnp
from jax import lax
from jax.experimental import pallas as pl
from jax.experimental.pallas import tpu as pltpu
from jax.experimental.pallas import tpu_sc as plsc

N = 10000
E = 160000
DIM = 128
EP = 163840
GW = 128
BN = 1000
BE = 1024
NHALF = N // 2
ACC_R = 5120
TRASH = 5100
ACCD_R = 656
ACCD_USED = 640
TRASH_D = 648
CH = 128


def _silu(v):
    return v * jax.nn.sigmoid(v)


def _ab_body(xh_ref, w1a_ref, w1b_ref, b1_ref, a_ref, b_ref):
    xh = xh_ref[...]
    a_ref[...] = jnp.dot(xh, w1a_ref[...], preferred_element_type=jnp.float32)
    b_ref[...] = (jnp.dot(xh, w1b_ref[...], preferred_element_type=jnp.float32)
                  + b1_ref[...])


def _stage_ab(xh2, w1a, w1b, b1):
    return pl.pallas_call(
        _ab_body,
        grid=(N // BN,),
        in_specs=[
            pl.BlockSpec((BN, DIM), lambda i: (i, 0)),
            pl.BlockSpec((DIM, DIM), lambda i: (0, 0)),
            pl.BlockSpec((DIM, DIM), lambda i: (0, 0)),
            pl.BlockSpec((1, DIM), lambda i: (0, 0)),
        ],
        out_specs=[
            pl.BlockSpec((BN, DIM), lambda i: (i, 0)),
            pl.BlockSpec((BN, DIM), lambda i: (i, 0)),
        ],
        out_shape=[
            jax.ShapeDtypeStruct((N, DIM), jnp.float32),
            jax.ShapeDtypeStruct((N, DIM), jnp.float32),
        ],
    )(xh2, w1a, w1b, b1)


def _stage_gather(a, b, x16, idx0, idx1):
    mesh = plsc.VectorSubcoreMesh(core_axis_name="core",
                                  subcore_axis_name="subcore")

    @functools.partial(
        pl.kernel,
        out_type=[
            jax.ShapeDtypeStruct((EP, DIM), jnp.float32),
            jax.ShapeDtypeStruct((EP, 16), jnp.float32),
        ],
        mesh=mesh,
        scratch_types=[
            pltpu.VMEM((GW, DIM), jnp.float32),
            pltpu.VMEM((GW, DIM), jnp.float32),
            pltpu.VMEM((GW, DIM), jnp.float32),
        ],
    )
    def k(a_hbm, b_hbm, x_hbm, i0_hbm, i1_hbm, z_hbm, rd_hbm, ga, gb, r0):
        def body(i0_v, i1_v, z_v, rd_v):
            pltpu.sync_copy(x_hbm.at[i0_v.at[0]], r0)

            @pl.loop(0, GW)
            def _(r):
                rd_v[r, :] = r0[r, pl.ds(0, 16)]

            pltpu.sync_copy(a_hbm.at[i0_v.at[0]], ga)
            pltpu.sync_copy(b_hbm.at[i1_v.at[0]], gb)
            pltpu.sync_copy(x_hbm.at[i1_v.at[0]], r0)

            @pl.loop(0, GW)
            def _(r):
                for c in range(DIM // 16):
                    sl = pl.ds(c * 16, 16)
                    z_v[r, sl] = ga[r, sl] + gb[r, sl]
                rd_v[r, :] = rd_v[r, :] - r0[r, pl.ds(0, 16)]

        pltpu.emit_pipeline(
            body,
            grid=(EP // GW,),
            in_specs=[
                pl.BlockSpec((1, GW), lambda i: (0, i)),
                pl.BlockSpec((1, GW), lambda i: (0, i)),
            ],
            out_specs=[
                pl.BlockSpec((GW, DIM), lambda i: (i, 0)),
                pl.BlockSpec((GW, 16), lambda i: (i, 0)),
            ],
            core_axis_name=("core", "subcore"),
            dimension_semantics=(pltpu.PARALLEL,),
        )(i0_hbm, i1_hbm, z_hbm, rd_hbm)

    return k(a, b, x16, idx0, idx1)


def _edge_body(z_ref, rd_ref, w0_ref, idx_ref, idf_ref, p0_ref, p1_ref,
               p2_ref, pd_ref, ie_ref, ied_ref):
    z = z_ref[...]
    rd = rd_ref[...]
    qsq = jnp.sum(rd * rd, axis=1, keepdims=True)
    zz = z + qsq * w0_ref[...]
    h = _silu(zz)
    rinv = lax.rsqrt(qsq)
    d = rd * rinv
    p0_ref[...] = h * d[:, 0:1]
    p1_ref[...] = h * d[:, 1:2]
    p2_ref[...] = h * d[:, 2:3]
    idc = idf_ref[...][:, 0:1].astype(jnp.int32)
    and7 = jnp.bitwise_and(idc, 7)
    lane = lax.broadcasted_iota(jnp.int32, (z.shape[0], DIM), 1)
    grp = lane >> 4
    sub = jnp.bitwise_and(lane, 15)
    val = jnp.where(sub == 0, d[:, 0:1],
                    jnp.where(sub == 1, d[:, 1:2],
                              jnp.where(sub == 2, d[:, 2:3], 1.0)))
    pd_ref[...] = jnp.where((grp == and7) & (sub < 4), val, 0.0)
    idx = idx_ref[...]
    in0 = (idx >= 0) & (idx < NHALF)
    in1 = (idx >= NHALF) & (idx < N)
    l0 = jnp.where(in0, idx, TRASH)
    l1 = jnp.where(in1, idx - NHALF, TRASH)
    ie_ref[0] = l0
    ie_ref[1] = l1
    ied_ref[0] = jnp.where(in0, l0 >> 3, TRASH_D)
    ied_ref[1] = jnp.where(in1, l1 >> 3, TRASH_D)


def _stage_edge(z, rd, w0, idx0s, idxf):
    ieb = pl.BlockSpec((2, BE // 128, 128), lambda i: (0, i, 0))
    ies = jax.ShapeDtypeStruct((2, EP // 128, 128), jnp.int32)
    return pl.pallas_call(
        _edge_body,
        grid=(EP // BE,),
        in_specs=[
            pl.BlockSpec((BE, DIM), lambda i: (i, 0)),
            pl.BlockSpec((BE, 16), lambda i: (i, 0)),
            pl.BlockSpec((1, DIM), lambda i: (0, 0)),
            pl.BlockSpec((BE // 128, 128), lambda i: (i, 0)),
            pl.BlockSpec((BE, 16), lambda i: (i, 0)),
        ],
        out_specs=[
            pl.BlockSpec((BE, DIM), lambda i: (i, 0)),
            pl.BlockSpec((BE, DIM), lambda i: (i, 0)),
            pl.BlockSpec((BE, DIM), lambda i: (i, 0)),
            pl.BlockSpec((BE, DIM), lambda i: (i, 0)),
            ieb, ieb,
        ],
        out_shape=[
            jax.ShapeDtypeStruct((EP, DIM), jnp.float32),
            jax.ShapeDtypeStruct((EP, DIM), jnp.float32),
            jax.ShapeDtypeStruct((EP, DIM), jnp.float32),
            jax.ShapeDtypeStruct((EP, DIM), jnp.float32),
            ies, ies,
        ],
    )(z, rd, w0, idx0s, idxf)


def _stage_scatter(p0, p1, p2, pd, ie, ied):
    mesh = plsc.VectorSubcoreMesh(core_axis_name="core",
                                  subcore_axis_name="subcore")
    n_chunks = EP // (16 * CH)

    @functools.partial(
        pl.kernel,
        out_type=[
            jax.ShapeDtypeStruct((N, DIM), jnp.float32),
            jax.ShapeDtypeStruct((N, DIM), jnp.float32),
            jax.ShapeDtypeStruct((N, DIM), jnp.float32),
            jax.ShapeDtypeStruct((2 * ACCD_USED, DIM), jnp.float32),
        ],
        mesh=mesh,
        scratch_types=[
            pltpu.VMEM((CH, DIM), jnp.float32),
            pltpu.VMEM((CH, DIM), jnp.float32),
            pltpu.VMEM((1, CH), jnp.int32),
            pltpu.VMEM((1, CH), jnp.int32),
            pltpu.VMEM((16, DIM), jnp.float32),
            pltpu.VMEM((40, DIM), jnp.float32),
            pltpu.VMEM_SHARED((ACC_R, DIM), jnp.float32),
            pltpu.VMEM_SHARED((ACCD_R, DIM), jnp.float32),
        ],
    )
    def k(p0_h, p1_h, p2_h, pd_h, ie_h, ied_h, s0_h, s1_h, s2_h, sd_h,
          pb, pdb, ib, ib2, zb, db, acc, accd):
        c = lax.axis_index("core")
        s = lax.axis_index("subcore")

        @pl.loop(0, 16)
        def _(r):
            for cc in range(DIM // 16):
                zb[r, pl.ds(cc * 16, 16)] = jnp.zeros((16,), jnp.float32)

        for kpass, (p_h, s_h) in enumerate(
                ((p0_h, s0_h), (p1_h, s1_h), (p2_h, s2_h))):

            @pl.loop(0, 20)
            def _(t):
                ro = s * 320 + t * 16
                pltpu.sync_copy(zb, acc.at[pl.ds(ro, 16)])

            if kpass == 0:
                @pl.loop(0, 2)
                def _(t):
                    ro = s * 41 + t * 16
                    pltpu.sync_copy(zb, accd.at[pl.ds(ro, 16)])

                pltpu.sync_copy(zb.at[pl.ds(0, 9)],
                                accd.at[pl.ds(s * 41 + 32, 9)])

            plsc.subcore_barrier()

            @pl.loop(0, n_chunks)
            def _(t):
                base = (s * n_chunks + t) * CH
                pltpu.sync_copy(ie_h.at[c, pl.ds(base, CH)], ib.at[0])
                pltpu.sync_copy(p_h.at[pl.ds(base, CH)], pb)
                pltpu.sync_copy(pb, acc.at[ib.at[0]], add=True)
                if kpass == 0:
                    pltpu.sync_copy(ied_h.at[c, pl.ds(base, CH)], ib2.at[0])
                    pltpu.sync_copy(pd_h.at[pl.ds(base, CH)], pdb)
                    pltpu.sync_copy(pdb, accd.at[ib2.at[0]], add=True)

            plsc.subcore_barrier()

            @pl.loop(0, 8)
            def _(t):
                cidx = s + t * 16

                @pl.when(cidx < NHALF // 40)
                def _():
                    ro = cidx * 40
                    pltpu.sync_copy(acc.at[pl.ds(ro, 40)], db)
                    pltpu.sync_copy(db, s_h.at[pl.ds(c * NHALF + ro, 40)])

            plsc.subcore_barrier()

        pltpu.sync_copy(accd.at[pl.ds(s * 40, 40)], db)
        pltpu.sync_copy(db, sd_h.at[pl.ds(c * ACCD_USED + s * 40, 40)])

    ief = ie.reshape(2, EP)
    iedf = ied.reshape(2, EP)
    return k(p0, p1, p2, pd, ief, iedf)


def _final_body(xh_ref, s0_ref, s1_ref, s2_ref, sd_ref, w2_ref, b2_ref,
                n1w1_ref, n1b1_ref, n1w2_ref, n1b2_ref,
                n2w1_ref, n2b1_ref, n2w2_ref, n2b2_ref,
                n3w1a_ref, n3w1b_ref, n3b1_ref, n3w2_ref, n3b2_ref,
                xo_ref, nh_ref, nrm_ref):
    xh = xh_ref[...]
    sd = sd_ref[...]
    deg = sd[:, 3:4]
    inv = 1.0 / jnp.maximum(deg, 1.0)
    w2 = w2_ref[...]
    b2 = b2_ref[...]
    means = []
    for k, s_ref in enumerate((s0_ref, s1_ref, s2_ref)):
        sk = s_ref[...]
        mk = (jnp.dot(sk, w2, preferred_element_type=jnp.float32)
              + b2 * sd[:, k:k+1]) * inv
        means.append(mk)
    nrm = jnp.sqrt(means[0] * means[0] + means[1] * means[1]
                   + means[2] * means[2])
    nrm_ref[...] = nrm
    t1 = _silu(jnp.dot(xh, n1w1_ref[...], preferred_element_type=jnp.float32)
               + n1b1_ref[...])
    o1 = jnp.dot(t1, n1w2_ref[...], preferred_element_type=jnp.float32) + n1b2_ref[...]
    cols = []
    for k in range(3):
        u = _silu(jnp.dot(means[k], n2w1_ref[...],
                          preferred_element_type=jnp.float32) + n2b1_ref[...])
        cols.append(jnp.dot(u, n2w2_ref[...],
                            preferred_element_type=jnp.float32) + n2b2_ref[...])
    xo_ref[...] = o1 + jnp.concatenate(cols, axis=1)
    g = _silu(jnp.dot(xh, n3w1a_ref[...], preferred_element_type=jnp.float32)
              + jnp.dot(nrm, n3w1b_ref[...], preferred_element_type=jnp.float32)
              + n3b1_ref[...])
    nh_ref[...] = (jnp.dot(g, n3w2_ref[...], preferred_element_type=jnp.float32)
                   + n3b2_ref[...])


def _stage_final(xh2, s0, s1, s2, sd, w2, b2, n1w1, n1b1, n1w2, n1b2,
                 n2w1, n2b1, n2w2, n2b2, n3w1a, n3w1b, n3b1, n3w2, n3b2):
    full = lambda r, c: pl.BlockSpec((r, c), lambda i: (0, 0))
    blk = lambda c: pl.BlockSpec((BN, c), lambda i: (i, 0))
    return pl.pallas_call(
        _final_body,
        grid=(N // BN,),
        in_specs=[
            blk(DIM), blk(DIM), blk(DIM), blk(DIM), blk(16),
            full(DIM, DIM), full(1, DIM),
            full(DIM, DIM), full(1, DIM), full(DIM, 3), full(1, 3),
            full(DIM, DIM), full(1, DIM), full(DIM, 1), full(1, 1),
            full(DIM, DIM), full(DIM, DIM), full(1, DIM), full(DIM, DIM),
            full(1, DIM),
        ],
        out_specs=[blk(3), blk(DIM), blk(DIM)],
        out_shape=[
            jax.ShapeDtypeStruct((N, 3), jnp.float32),
            jax.ShapeDtypeStruct((N, DIM), jnp.float32),
            jax.ShapeDtypeStruct((N, DIM), jnp.float32),
        ],
    )(xh2, s0, s1, s2, sd, w2, b2, n1w1, n1b1, n1w2, n1b2,
      n2w1, n2b1, n2w2, n2b2, n3w1a, n3w1b, n3b1, n3w2, n3b2)


def kernel(x, xh, e, sc_W1, sc_b1, sc_W2, sc_b2, n1_W1, n1_b1, n1_W2, n1_b2,
           n2_W1, n2_b1, n2_W2, n2_b2, n3_W1, n3_b1, n3_W2, n3_b2):
    x2 = x[0]
    xh2 = xh[0]
    w0 = sc_W1[0:1]
    w1a = sc_W1[1:1+DIM]
    w1b = sc_W1[1+DIM:]
    b1 = sc_b1.reshape(1, DIM)

    a, b = _stage_ab(xh2, w1a, w1b, b1)

    x16 = jnp.pad(x2, ((0, 0), (0, 125)))
    pad = EP - E
    idx0 = e[0]
    idx1 = e[1]
    idx0g = jnp.pad(idx0, (0, pad)).reshape(1, EP)
    idx1g = jnp.pad(idx1, (0, pad)).reshape(1, EP)
    idx0p = jnp.pad(idx0, (0, pad), constant_values=-1)
    idx0s = idx0p.reshape(EP // 128, 128)
    idxf = jnp.pad(idx0p.astype(jnp.float32)[:, None], ((0, 0), (0, 15)))

    z, rd = _stage_gather(a, b, x16, idx0g, idx1g)
    p0, p1, p2, pd, ie, ied = _stage_edge(z, rd, w0, idx0s, idxf)
    s0, s1, s2, sd2 = _stage_scatter(p0, p1, p2, pd, ie, ied)
    sd = jnp.concatenate(
        [sd2[c * ACCD_USED:(c + 1) * ACCD_USED].reshape(8 * ACCD_USED, 16)[:NHALF]
         for c in range(2)], axis=0)

    xo, nh, nrm = _stage_final(
        xh2, s0, s1, s2, sd, sc_W2, sc_b2.reshape(1, DIM),
        n1_W1, n1_b1.reshape(1, DIM), n1_W2, n1_b2.reshape(1, 3),
        n2_W1, n2_b1.reshape(1, DIM), n2_W2, n2_b2.reshape(1, 1),
        n3_W1[:DIM], n3_W1[DIM:], n3_b1.reshape(1, DIM),
        n3_W2, n3_b2.reshape(1, DIM))

    return (xo[None], nh[None], nrm[None])


# speedup vs baseline: 18.5885x; 1.2872x over previous
import dataclasses
import functools

import jax
import jax.numpy as jnp
from jax import lax
from jax.experimental import pallas as pl
from jax.experimental.pallas import tpu as pltpu
from jax.experimental.pallas import tpu_sc as plsc

N = 10000
E = 160000
DIM = 128
EP = 163840
GW = 128
BN = 1000
BE = 1024
NHALF = N // 2
ACC_R = 5120
TRASH = 5100
ACCD_R = 656
ACCD_USED = 640
TRASH_D = 648
CH = 128


def _silu(v):
    return v * jax.nn.sigmoid(v)


def _sc_compiler_params():
    cp = pltpu.CompilerParams()
    if "needs_layout_passes" in pltpu.CompilerParams.__dataclass_fields__:
        cp = dataclasses.replace(cp, needs_layout_passes=False)
    return cp


def _ab_body(xh_ref, w1a_ref, w1b_ref, b1_ref, a_ref, b_ref):
    xh = xh_ref[...]
    a_ref[...] = jnp.dot(xh, w1a_ref[...], preferred_element_type=jnp.float32)
    b_ref[...] = (jnp.dot(xh, w1b_ref[...], preferred_element_type=jnp.float32)
                  + b1_ref[...])


def _stage_ab(xh2, w1a, w1b, b1):
    return pl.pallas_call(
        _ab_body,
        grid=(N // BN,),
        in_specs=[
            pl.BlockSpec((BN, DIM), lambda i: (i, 0)),
            pl.BlockSpec((DIM, DIM), lambda i: (0, 0)),
            pl.BlockSpec((DIM, DIM), lambda i: (0, 0)),
            pl.BlockSpec((1, DIM), lambda i: (0, 0)),
        ],
        out_specs=[
            pl.BlockSpec((BN, DIM), lambda i: (i, 0)),
            pl.BlockSpec((BN, DIM), lambda i: (i, 0)),
        ],
        out_shape=[
            jax.ShapeDtypeStruct((N, DIM), jnp.float32),
            jax.ShapeDtypeStruct((N, DIM), jnp.float32),
        ],
    )(xh2, w1a, w1b, b1)


XT_R = 235


def _stage_gather(a, b, xflat, idx0, idx1):
    mesh = plsc.VectorSubcoreMesh(core_axis_name="core",
                                  subcore_axis_name="subcore")

    @functools.partial(
        pl.kernel,
        out_type=[
            jax.ShapeDtypeStruct((EP, DIM), jnp.float32),
            jax.ShapeDtypeStruct((EP, 16), jnp.float32),
        ],
        mesh=mesh,
        scratch_types=[
            pltpu.VMEM((GW, DIM), jnp.float32),
            pltpu.VMEM((GW, DIM), jnp.float32),
            pltpu.VMEM((XT_R, 128), jnp.float32),
        ],
        compiler_params=_sc_compiler_params(),
    )
    def k(a_hbm, b_hbm, x_hbm, i0_hbm, i1_hbm, z_hbm, rd_hbm, ga, gb, xt):
        pltpu.sync_copy(x_hbm, xt)

        def body(i0_v, i1_v, z_v, rd_v):
            pltpu.sync_copy(a_hbm.at[i0_v.at[0]], ga)
            pltpu.sync_copy(b_hbm.at[i1_v.at[0]], gb)

            @pl.loop(0, GW // 16)
            def _(g):
                i0 = i0_v[0, pl.ds(g * 16, 16)]
                i1 = i1_v[0, pl.ds(g * 16, 16)]
                rows = lax.iota(jnp.int32, 16) + g * 16
                for c in range(3):
                    f0 = i0 * 3 + c
                    f1 = i1 * 3 + c
                    v0 = plsc.load_gather(xt, [f0 >> 7, f0 & 127])
                    v1 = plsc.load_gather(xt, [f1 >> 7, f1 & 127])
                    plsc.store_scatter(rd_v, [rows, jnp.full((16,), c, jnp.int32)],
                                       v0 - v1)

            @pl.loop(0, GW)
            def _(r):
                for c in range(DIM // 16):
                    sl = pl.ds(c * 16, 16)
                    z_v[r, sl] = ga[r, sl] + gb[r, sl]

        pltpu.emit_pipeline(
            body,
            grid=(EP // GW,),
            in_specs=[
                pl.BlockSpec((1, GW), lambda i: (0, i)),
                pl.BlockSpec((1, GW), lambda i: (0, i)),
            ],
            out_specs=[
                pl.BlockSpec((GW, DIM), lambda i: (i, 0)),
                pl.BlockSpec((GW, 16), lambda i: (i, 0)),
            ],
            core_axis_name=("core", "subcore"),
            dimension_semantics=(pltpu.PARALLEL,),
        )(i0_hbm, i1_hbm, z_hbm, rd_hbm)

    return k(a, b, xflat, idx0, idx1)


def _edge_body(z_ref, rd_ref, w0_ref, idx_ref, idf_ref, p0_ref, p1_ref,
               p2_ref, pd_ref, ie_ref, ied_ref):
    z = z_ref[...]
    rd = rd_ref[...][:, 0:3]
    qsq = jnp.sum(rd * rd, axis=1, keepdims=True)
    zz = z + qsq * w0_ref[...]
    h = _silu(zz)
    rinv = lax.rsqrt(qsq)
    d = rd * rinv
    p0_ref[...] = h * d[:, 0:1]
    p1_ref[...] = h * d[:, 1:2]
    p2_ref[...] = h * d[:, 2:3]
    idc = idf_ref[...][:, 0:1].astype(jnp.int32)
    and7 = jnp.bitwise_and(idc, 7)
    lane = lax.broadcasted_iota(jnp.int32, (z.shape[0], DIM), 1)
    grp = lane >> 4
    sub = jnp.bitwise_and(lane, 15)
    val = jnp.where(sub == 0, d[:, 0:1],
                    jnp.where(sub == 1, d[:, 1:2],
                              jnp.where(sub == 2, d[:, 2:3], 1.0)))
    pd_ref[...] = jnp.where((grp == and7) & (sub < 4), val, 0.0)
    idx = idx_ref[...]
    in0 = (idx >= 0) & (idx < NHALF)
    in1 = (idx >= NHALF) & (idx < N)
    l0 = jnp.where(in0, idx, TRASH)
    l1 = jnp.where(in1, idx - NHALF, TRASH)
    ie_ref[0] = l0
    ie_ref[1] = l1
    ied_ref[0] = jnp.where(in0, l0 >> 3, TRASH_D)
    ied_ref[1] = jnp.where(in1, l1 >> 3, TRASH_D)


def _stage_edge(z, rd, w0, idx0s, idxf):
    ieb = pl.BlockSpec((2, BE // 128, 128), lambda i: (0, i, 0))
    ies = jax.ShapeDtypeStruct((2, EP // 128, 128), jnp.int32)
    return pl.pallas_call(
        _edge_body,
        grid=(EP // BE,),
        in_specs=[
            pl.BlockSpec((BE, DIM), lambda i: (i, 0)),
            pl.BlockSpec((BE, 16), lambda i: (i, 0)),
            pl.BlockSpec((1, DIM), lambda i: (0, 0)),
            pl.BlockSpec((BE // 128, 128), lambda i: (i, 0)),
            pl.BlockSpec((BE, 16), lambda i: (i, 0)),
        ],
        out_specs=[
            pl.BlockSpec((BE, DIM), lambda i: (i, 0)),
            pl.BlockSpec((BE, DIM), lambda i: (i, 0)),
            pl.BlockSpec((BE, DIM), lambda i: (i, 0)),
            pl.BlockSpec((BE, DIM), lambda i: (i, 0)),
            ieb, ieb,
        ],
        out_shape=[
            jax.ShapeDtypeStruct((EP, DIM), jnp.float32),
            jax.ShapeDtypeStruct((EP, DIM), jnp.float32),
            jax.ShapeDtypeStruct((EP, DIM), jnp.float32),
            jax.ShapeDtypeStruct((EP, DIM), jnp.float32),
            ies, ies,
        ],
    )(z, rd, w0, idx0s, idxf)


def _stage_scatter(p0, p1, p2, pd, ie, ied):
    mesh = plsc.VectorSubcoreMesh(core_axis_name="core",
                                  subcore_axis_name="subcore")
    n_chunks = EP // (16 * CH)

    @functools.partial(
        pl.kernel,
        out_type=[
            jax.ShapeDtypeStruct((N, DIM), jnp.float32),
            jax.ShapeDtypeStruct((N, DIM), jnp.float32),
            jax.ShapeDtypeStruct((N, DIM), jnp.float32),
            jax.ShapeDtypeStruct((2 * ACCD_USED, DIM), jnp.float32),
        ],
        mesh=mesh,
        scratch_types=[
            pltpu.VMEM((CH, DIM), jnp.float32),
            pltpu.VMEM((CH, DIM), jnp.float32),
            pltpu.VMEM((1, CH), jnp.int32),
            pltpu.VMEM((1, CH), jnp.int32),
            pltpu.VMEM((16, DIM), jnp.float32),
            pltpu.VMEM((40, DIM), jnp.float32),
            pltpu.VMEM_SHARED((ACC_R, DIM), jnp.float32),
            pltpu.VMEM_SHARED((ACCD_R, DIM), jnp.float32),
        ],
    )
    def k(p0_h, p1_h, p2_h, pd_h, ie_h, ied_h, s0_h, s1_h, s2_h, sd_h,
          pb, pdb, ib, ib2, zb, db, acc, accd):
        c = lax.axis_index("core")
        s = lax.axis_index("subcore")

        @pl.loop(0, 16)
        def _(r):
            for cc in range(DIM // 16):
                zb[r, pl.ds(cc * 16, 16)] = jnp.zeros((16,), jnp.float32)

        for kpass, (p_h, s_h) in enumerate(
                ((p0_h, s0_h), (p1_h, s1_h), (p2_h, s2_h))):

            @pl.loop(0, 20)
            def _(t):
                ro = s * 320 + t * 16
                pltpu.sync_copy(zb, acc.at[pl.ds(ro, 16)])

            if kpass == 0:
                @pl.loop(0, 2)
                def _(t):
                    ro = s * 41 + t * 16
                    pltpu.sync_copy(zb, accd.at[pl.ds(ro, 16)])

                pltpu.sync_copy(zb.at[pl.ds(0, 9)],
                                accd.at[pl.ds(s * 41 + 32, 9)])

            plsc.subcore_barrier()

            @pl.loop(0, n_chunks)
            def _(t):
                base = (s * n_chunks + t) * CH
                pltpu.sync_copy(ie_h.at[c, pl.ds(base, CH)], ib.at[0])
                pltpu.sync_copy(p_h.at[pl.ds(base, CH)], pb)
                pltpu.sync_copy(pb, acc.at[ib.at[0]], add=True)
                if kpass == 0:
                    pltpu.sync_copy(ied_h.at[c, pl.ds(base, CH)], ib2.at[0])
                    pltpu.sync_copy(pd_h.at[pl.ds(base, CH)], pdb)
                    pltpu.sync_copy(pdb, accd.at[ib2.at[0]], add=True)

            plsc.subcore_barrier()

            @pl.loop(0, 8)
            def _(t):
                cidx = s + t * 16

                @pl.when(cidx < NHALF // 40)
                def _():
                    ro = cidx * 40
                    pltpu.sync_copy(acc.at[pl.ds(ro, 40)], db)
                    pltpu.sync_copy(db, s_h.at[pl.ds(c * NHALF + ro, 40)])

            plsc.subcore_barrier()

        pltpu.sync_copy(accd.at[pl.ds(s * 40, 40)], db)
        pltpu.sync_copy(db, sd_h.at[pl.ds(c * ACCD_USED + s * 40, 40)])

    ief = ie.reshape(2, EP)
    iedf = ied.reshape(2, EP)
    return k(p0, p1, p2, pd, ief, iedf)


def _final_body(xh_ref, s0_ref, s1_ref, s2_ref, sd_ref, w2_ref, b2_ref,
                n1w1_ref, n1b1_ref, n1w2_ref, n1b2_ref,
                n2w1_ref, n2b1_ref, n2w2_ref, n2b2_ref,
                n3w1a_ref, n3w1b_ref, n3b1_ref, n3w2_ref, n3b2_ref,
                xo_ref, nh_ref, nrm_ref):
    xh = xh_ref[...]
    sd = sd_ref[...]
    deg = sd[:, 3:4]
    inv = 1.0 / jnp.maximum(deg, 1.0)
    w2 = w2_ref[...]
    b2 = b2_ref[...]
    means = []
    for k, s_ref in enumerate((s0_ref, s1_ref, s2_ref)):
        sk = s_ref[...]
        mk = (jnp.dot(sk, w2, preferred_element_type=jnp.float32)
              + b2 * sd[:, k:k+1]) * inv
        means.append(mk)
    nrm = jnp.sqrt(means[0] * means[0] + means[1] * means[1]
                   + means[2] * means[2])
    nrm_ref[...] = nrm
    t1 = _silu(jnp.dot(xh, n1w1_ref[...], preferred_element_type=jnp.float32)
               + n1b1_ref[...])
    o1 = jnp.dot(t1, n1w2_ref[...], preferred_element_type=jnp.float32) + n1b2_ref[...]
    cols = []
    for k in range(3):
        u = _silu(jnp.dot(means[k], n2w1_ref[...],
                          preferred_element_type=jnp.float32) + n2b1_ref[...])
        cols.append(jnp.dot(u, n2w2_ref[...],
                            preferred_element_type=jnp.float32) + n2b2_ref[...])
    xo_ref[...] = o1 + jnp.concatenate(cols, axis=1)
    g = _silu(jnp.dot(xh, n3w1a_ref[...], preferred_element_type=jnp.float32)
              + jnp.dot(nrm, n3w1b_ref[...], preferred_element_type=jnp.float32)
              + n3b1_ref[...])
    nh_ref[...] = (jnp.dot(g, n3w2_ref[...], preferred_element_type=jnp.float32)
                   + n3b2_ref[...])


def _stage_final(xh2, s0, s1, s2, sd, w2, b2, n1w1, n1b1, n1w2, n1b2,
                 n2w1, n2b1, n2w2, n2b2, n3w1a, n3w1b, n3b1, n3w2, n3b2):
    full = lambda r, c: pl.BlockSpec((r, c), lambda i: (0, 0))
    blk = lambda c: pl.BlockSpec((BN, c), lambda i: (i, 0))
    return pl.pallas_call(
        _final_body,
        grid=(N // BN,),
        in_specs=[
            blk(DIM), blk(DIM), blk(DIM), blk(DIM), blk(16),
            full(DIM, DIM), full(1, DIM),
            full(DIM, DIM), full(1, DIM), full(DIM, 3), full(1, 3),
            full(DIM, DIM), full(1, DIM), full(DIM, 1), full(1, 1),
            full(DIM, DIM), full(DIM, DIM), full(1, DIM), full(DIM, DIM),
            full(1, DIM),
        ],
        out_specs=[blk(3), blk(DIM), blk(DIM)],
        out_shape=[
            jax.ShapeDtypeStruct((N, 3), jnp.float32),
            jax.ShapeDtypeStruct((N, DIM), jnp.float32),
            jax.ShapeDtypeStruct((N, DIM), jnp.float32),
        ],
    )(xh2, s0, s1, s2, sd, w2, b2, n1w1, n1b1, n1w2, n1b2,
      n2w1, n2b1, n2w2, n2b2, n3w1a, n3w1b, n3b1, n3w2, n3b2)


def kernel(x, xh, e, sc_W1, sc_b1, sc_W2, sc_b2, n1_W1, n1_b1, n1_W2, n1_b2,
           n2_W1, n2_b1, n2_W2, n2_b2, n3_W1, n3_b1, n3_W2, n3_b2):
    x2 = x[0]
    xh2 = xh[0]
    w0 = sc_W1[0:1]
    w1a = sc_W1[1:1+DIM]
    w1b = sc_W1[1+DIM:]
    b1 = sc_b1.reshape(1, DIM)

    a, b = _stage_ab(xh2, w1a, w1b, b1)

    xflat = jnp.pad(x2.reshape(3 * N), (0, XT_R * 128 - 3 * N)).reshape(XT_R, 128)
    pad = EP - E
    idx0 = e[0]
    idx1 = e[1]
    idx0g = jnp.pad(idx0, (0, pad)).reshape(1, EP)
    idx1g = jnp.pad(idx1, (0, pad)).reshape(1, EP)
    idx0p = jnp.pad(idx0, (0, pad), constant_values=-1)
    idx0s = idx0p.reshape(EP // 128, 128)
    idxf = jnp.pad(idx0p.astype(jnp.float32)[:, None], ((0, 0), (0, 15)))

    z, rd = _stage_gather(a, b, xflat, idx0g, idx1g)
    p0, p1, p2, pd, ie, ied = _stage_edge(z, rd, w0, idx0s, idxf)
    s0, s1, s2, sd2 = _stage_scatter(p0, p1, p2, pd, ie, ied)
    sd = jnp.concatenate(
        [sd2[c * ACCD_USED:(c + 1) * ACCD_USED].reshape(8 * ACCD_USED, 16)[:NHALF]
         for c in range(2)], axis=0)

    xo, nh, nrm = _stage_final(
        xh2, s0, s1, s2, sd, sc_W2, sc_b2.reshape(1, DIM),
        n1_W1, n1_b1.reshape(1, DIM), n1_W2, n1_b2.reshape(1, 3),
        n2_W1, n2_b1.reshape(1, DIM), n2_W2, n2_b2.reshape(1, 1),
        n3_W1[:DIM], n3_W1[DIM:], n3_b1.reshape(1, DIM),
        n3_W2, n3_b2.reshape(1, DIM))

    return (xo[None], nh[None], nrm[None])


# speedup vs baseline: 24.7085x; 1.3292x over previous
import dataclasses
import functools

import jax
import jax.numpy as jnp
from jax import lax
from jax.experimental import pallas as pl
from jax.experimental.pallas import tpu as pltpu
from jax.experimental.pallas import tpu_sc as plsc

N = 10000
E = 160000
DIM = 128
EP = 163840
GW = 128
BN = 1000
BE = 1024
NHALF = N // 2
ACC_R = 5120
TRASH = 5100
ACCD_R = 656
ACCD_USED = 640
TRASH_D = 648
CH = 128


def _silu(v):
    return v * jax.nn.sigmoid(v)


def _sc_compiler_params():
    cp = pltpu.CompilerParams()
    if "needs_layout_passes" in pltpu.CompilerParams.__dataclass_fields__:
        cp = dataclasses.replace(cp, needs_layout_passes=False)
    return cp


def _ab_body(xh_ref, w1a_ref, w1b_ref, b1_ref, a_ref, b_ref):
    xh = xh_ref[...]
    a_ref[...] = jnp.dot(xh, w1a_ref[...], preferred_element_type=jnp.float32)
    b_ref[...] = (jnp.dot(xh, w1b_ref[...], preferred_element_type=jnp.float32)
                  + b1_ref[...])


def _stage_ab(xh2, w1a, w1b, b1):
    return pl.pallas_call(
        _ab_body,
        grid=(N // BN,),
        in_specs=[
            pl.BlockSpec((BN, DIM), lambda i: (i, 0)),
            pl.BlockSpec((DIM, DIM), lambda i: (0, 0)),
            pl.BlockSpec((DIM, DIM), lambda i: (0, 0)),
            pl.BlockSpec((1, DIM), lambda i: (0, 0)),
        ],
        out_specs=[
            pl.BlockSpec((BN, DIM), lambda i: (i, 0)),
            pl.BlockSpec((BN, DIM), lambda i: (i, 0)),
        ],
        out_shape=[
            jax.ShapeDtypeStruct((N, DIM), jnp.float32),
            jax.ShapeDtypeStruct((N, DIM), jnp.float32),
        ],
    )(xh2, w1a, w1b, b1)


XT_R = 235


def _stage_gather(a, b, xflat, idx0, idx1):
    mesh = plsc.VectorSubcoreMesh(core_axis_name="core",
                                  subcore_axis_name="subcore")

    @functools.partial(
        pl.kernel,
        out_type=[
            jax.ShapeDtypeStruct((EP, DIM), jnp.float32),
            jax.ShapeDtypeStruct((EP, 16), jnp.float32),
        ],
        mesh=mesh,
        scratch_types=[
            pltpu.VMEM((GW, DIM), jnp.float32),
            pltpu.VMEM((GW, DIM), jnp.float32),
            pltpu.VMEM((XT_R, 128), jnp.float32),
            pltpu.SemaphoreType.DMA,
            pltpu.SemaphoreType.DMA,
        ],
        compiler_params=_sc_compiler_params(),
    )
    def k(a_hbm, b_hbm, x_hbm, i0_hbm, i1_hbm, z_hbm, rd_hbm, ga, gb, xt,
          sga, sgb):
        pltpu.sync_copy(x_hbm, xt)

        def body(i0_v, i1_v, z_v, rd_v):
            ca = pltpu.async_copy(a_hbm.at[i0_v.at[0]], ga, sga)
            cb = pltpu.async_copy(b_hbm.at[i1_v.at[0]], gb, sgb)

            @pl.loop(0, GW // 16)
            def _(g):
                i0 = i0_v[0, pl.ds(g * 16, 16)]
                i1 = i1_v[0, pl.ds(g * 16, 16)]
                rows = lax.iota(jnp.int32, 16) + g * 16
                for c in range(3):
                    f0 = i0 * 3 + c
                    f1 = i1 * 3 + c
                    v0 = plsc.load_gather(xt, [f0 >> 7, f0 & 127])
                    v1 = plsc.load_gather(xt, [f1 >> 7, f1 & 127])
                    plsc.store_scatter(rd_v, [rows, jnp.full((16,), c, jnp.int32)],
                                       v0 - v1)

            ca.wait()
            cb.wait()

            @pl.loop(0, GW)
            def _(r):
                for c in range(DIM // 16):
                    sl = pl.ds(c * 16, 16)
                    z_v[r, sl] = ga[r, sl] + gb[r, sl]

        pltpu.emit_pipeline(
            body,
            grid=(EP // GW,),
            in_specs=[
                pl.BlockSpec((1, GW), lambda i: (0, i)),
                pl.BlockSpec((1, GW), lambda i: (0, i)),
            ],
            out_specs=[
                pl.BlockSpec((GW, DIM), lambda i: (i, 0)),
                pl.BlockSpec((GW, 16), lambda i: (i, 0)),
            ],
            core_axis_name=("core", "subcore"),
            dimension_semantics=(pltpu.PARALLEL,),
        )(i0_hbm, i1_hbm, z_hbm, rd_hbm)

    return k(a, b, xflat, idx0, idx1)


def _edge_body(z_ref, rd_ref, w0_ref, idx_ref, idf_ref, p0_ref, p1_ref,
               p2_ref, pd_ref, ie_ref, ied_ref):
    z = z_ref[...]
    rd = rd_ref[...][:, 0:3]
    qsq = jnp.sum(rd * rd, axis=1, keepdims=True)
    zz = z + qsq * w0_ref[...]
    h = _silu(zz)
    rinv = lax.rsqrt(qsq)
    d = rd * rinv
    p0_ref[...] = h * d[:, 0:1]
    p1_ref[...] = h * d[:, 1:2]
    p2_ref[...] = h * d[:, 2:3]
    idc = idf_ref[...][:, 0:1].astype(jnp.int32)
    and7 = jnp.bitwise_and(idc, 7)
    lane = lax.broadcasted_iota(jnp.int32, (z.shape[0], DIM), 1)
    grp = lane >> 4
    sub = jnp.bitwise_and(lane, 15)
    val = jnp.where(sub == 0, d[:, 0:1],
                    jnp.where(sub == 1, d[:, 1:2],
                              jnp.where(sub == 2, d[:, 2:3], 1.0)))
    pd_ref[...] = jnp.where((grp == and7) & (sub < 4), val, 0.0)
    idx = idx_ref[...]
    in0 = (idx >= 0) & (idx < NHALF)
    in1 = (idx >= NHALF) & (idx < N)
    l0 = jnp.where(in0, idx, TRASH)
    l1 = jnp.where(in1, idx - NHALF, TRASH)
    ie_ref[0] = l0
    ie_ref[1] = l1
    ied_ref[0] = jnp.where(in0, l0 >> 3, TRASH_D)
    ied_ref[1] = jnp.where(in1, l1 >> 3, TRASH_D)


def _stage_edge(z, rd, w0, idx0s, idxf):
    ieb = pl.BlockSpec((2, BE // 128, 128), lambda i: (0, i, 0))
    ies = jax.ShapeDtypeStruct((2, EP // 128, 128), jnp.int32)
    return pl.pallas_call(
        _edge_body,
        grid=(EP // BE,),
        in_specs=[
            pl.BlockSpec((BE, DIM), lambda i: (i, 0)),
            pl.BlockSpec((BE, 16), lambda i: (i, 0)),
            pl.BlockSpec((1, DIM), lambda i: (0, 0)),
            pl.BlockSpec((BE // 128, 128), lambda i: (i, 0)),
            pl.BlockSpec((BE, 16), lambda i: (i, 0)),
        ],
        out_specs=[
            pl.BlockSpec((BE, DIM), lambda i: (i, 0)),
            pl.BlockSpec((BE, DIM), lambda i: (i, 0)),
            pl.BlockSpec((BE, DIM), lambda i: (i, 0)),
            pl.BlockSpec((BE, DIM), lambda i: (i, 0)),
            ieb, ieb,
        ],
        out_shape=[
            jax.ShapeDtypeStruct((EP, DIM), jnp.float32),
            jax.ShapeDtypeStruct((EP, DIM), jnp.float32),
            jax.ShapeDtypeStruct((EP, DIM), jnp.float32),
            jax.ShapeDtypeStruct((EP, DIM), jnp.float32),
            ies, ies,
        ],
    )(z, rd, w0, idx0s, idxf)


def _stage_scatter(p0, p1, p2, pd, ie, ied):
    mesh = plsc.VectorSubcoreMesh(core_axis_name="core",
                                  subcore_axis_name="subcore")
    n_chunks = EP // (16 * CH)

    @functools.partial(
        pl.kernel,
        out_type=[
            jax.ShapeDtypeStruct((N, DIM), jnp.float32),
            jax.ShapeDtypeStruct((N, DIM), jnp.float32),
            jax.ShapeDtypeStruct((N, DIM), jnp.float32),
            jax.ShapeDtypeStruct((2 * ACCD_USED, DIM), jnp.float32),
        ],
        mesh=mesh,
        scratch_types=[
            pltpu.VMEM((CH, DIM), jnp.float32),
            pltpu.VMEM((CH, DIM), jnp.float32),
            pltpu.VMEM((CH, DIM), jnp.float32),
            pltpu.VMEM((CH, DIM), jnp.float32),
            pltpu.VMEM((1, CH), jnp.int32),
            pltpu.VMEM((1, CH), jnp.int32),
            pltpu.VMEM((1, CH), jnp.int32),
            pltpu.VMEM((1, CH), jnp.int32),
            pltpu.VMEM((16, DIM), jnp.float32),
            pltpu.VMEM((40, DIM), jnp.float32),
            pltpu.VMEM_SHARED((ACC_R, DIM), jnp.float32),
            pltpu.VMEM_SHARED((ACCD_R, DIM), jnp.float32),
            pltpu.SemaphoreType.DMA,
            pltpu.SemaphoreType.DMA,
        ],
    )
    def k(p0_h, p1_h, p2_h, pd_h, ie_h, ied_h, s0_h, s1_h, s2_h, sd_h,
          pba, pbb, pdba, pdbb, iba, ibb, ib2a, ib2b, zb, db, acc, accd,
          sema, semb):
        c = lax.axis_index("core")
        s = lax.axis_index("subcore")

        @pl.loop(0, 16)
        def _(r):
            for cc in range(DIM // 16):
                zb[r, pl.ds(cc * 16, 16)] = jnp.zeros((16,), jnp.float32)

        for kpass, (p_h, s_h) in enumerate(
                ((p0_h, s0_h), (p1_h, s1_h), (p2_h, s2_h))):

            @pl.loop(0, 20)
            def _(t):
                ro = s * 320 + t * 16
                pltpu.sync_copy(zb, acc.at[pl.ds(ro, 16)])

            if kpass == 0:
                @pl.loop(0, 2)
                def _(t):
                    ro = s * 41 + t * 16
                    pltpu.sync_copy(zb, accd.at[pl.ds(ro, 16)])

                pltpu.sync_copy(zb.at[pl.ds(0, 9)],
                                accd.at[pl.ds(s * 41 + 32, 9)])

            plsc.subcore_barrier()

            def issue(t, pb_, pdb_, ib_, ib2_, sem):
                base = (s * n_chunks + t) * CH
                pltpu.async_copy(ie_h.at[c, pl.ds(base, CH)], ib_.at[0], sem)
                pltpu.async_copy(p_h.at[pl.ds(base, CH)], pb_, sem)
                if kpass == 0:
                    pltpu.async_copy(ied_h.at[c, pl.ds(base, CH)], ib2_.at[0],
                                     sem)
                    pltpu.async_copy(pd_h.at[pl.ds(base, CH)], pdb_, sem)

            def drain(t, pb_, pdb_, ib_, ib2_, sem):
                base = (s * n_chunks + t) * CH
                pltpu.make_async_copy(ie_h.at[c, pl.ds(base, CH)], ib_.at[0],
                                      sem).wait()
                pltpu.make_async_copy(p_h.at[pl.ds(base, CH)], pb_, sem).wait()
                if kpass == 0:
                    pltpu.make_async_copy(ied_h.at[c, pl.ds(base, CH)],
                                          ib2_.at[0], sem).wait()
                    pltpu.make_async_copy(pd_h.at[pl.ds(base, CH)], pdb_,
                                          sem).wait()

            def scat(pb_, pdb_, ib_, ib2_):
                pltpu.sync_copy(pb_, acc.at[ib_.at[0]], add=True)
                if kpass == 0:
                    pltpu.sync_copy(pdb_, accd.at[ib2_.at[0]], add=True)

            seta = (pba, pdba, iba, ib2a, sema)
            setb = (pbb, pdbb, ibb, ib2b, semb)
            issue(0, *seta)

            @pl.loop(0, n_chunks, step=2)
            def _(t):
                issue(t + 1, *setb)
                drain(t, *seta)
                scat(*seta[:4])

                @pl.when(t + 2 < n_chunks)
                def _():
                    issue(t + 2, *seta)

                drain(t + 1, *setb)
                scat(*setb[:4])

            plsc.subcore_barrier()

            @pl.loop(0, 8)
            def _(t):
                cidx = s + t * 16

                @pl.when(cidx < NHALF // 40)
                def _():
                    ro = cidx * 40
                    pltpu.sync_copy(acc.at[pl.ds(ro, 40)], db)
                    pltpu.sync_copy(db, s_h.at[pl.ds(c * NHALF + ro, 40)])

            plsc.subcore_barrier()

        pltpu.sync_copy(accd.at[pl.ds(s * 40, 40)], db)
        pltpu.sync_copy(db, sd_h.at[pl.ds(c * ACCD_USED + s * 40, 40)])

    ief = ie.reshape(2, EP)
    iedf = ied.reshape(2, EP)
    return k(p0, p1, p2, pd, ief, iedf)


def _final_body(xh_ref, s0_ref, s1_ref, s2_ref, sd_ref, w2_ref, b2_ref,
                n1w1_ref, n1b1_ref, n1w2_ref, n1b2_ref,
                n2w1_ref, n2b1_ref, n2w2_ref, n2b2_ref,
                n3w1a_ref, n3w1b_ref, n3b1_ref, n3w2_ref, n3b2_ref,
                xo_ref, nh_ref, nrm_ref):
    xh = xh_ref[...]
    sd = sd_ref[...]
    deg = sd[:, 3:4]
    inv = 1.0 / jnp.maximum(deg, 1.0)
    w2 = w2_ref[...]
    b2 = b2_ref[...]
    means = []
    for k, s_ref in enumerate((s0_ref, s1_ref, s2_ref)):
        sk = s_ref[...]
        mk = (jnp.dot(sk, w2, preferred_element_type=jnp.float32)
              + b2 * sd[:, k:k+1]) * inv
        means.append(mk)
    nrm = jnp.sqrt(means[0] * means[0] + means[1] * means[1]
                   + means[2] * means[2])
    nrm_ref[...] = nrm
    t1 = _silu(jnp.dot(xh, n1w1_ref[...], preferred_element_type=jnp.float32)
               + n1b1_ref[...])
    o1 = jnp.dot(t1, n1w2_ref[...], preferred_element_type=jnp.float32) + n1b2_ref[...]
    cols = []
    for k in range(3):
        u = _silu(jnp.dot(means[k], n2w1_ref[...],
                          preferred_element_type=jnp.float32) + n2b1_ref[...])
        cols.append(jnp.dot(u, n2w2_ref[...],
                            preferred_element_type=jnp.float32) + n2b2_ref[...])
    xo_ref[...] = o1 + jnp.concatenate(cols, axis=1)
    g = _silu(jnp.dot(xh, n3w1a_ref[...], preferred_element_type=jnp.float32)
              + jnp.dot(nrm, n3w1b_ref[...], preferred_element_type=jnp.float32)
              + n3b1_ref[...])
    nh_ref[...] = (jnp.dot(g, n3w2_ref[...], preferred_element_type=jnp.float32)
                   + n3b2_ref[...])


def _stage_final(xh2, s0, s1, s2, sd, w2, b2, n1w1, n1b1, n1w2, n1b2,
                 n2w1, n2b1, n2w2, n2b2, n3w1a, n3w1b, n3b1, n3w2, n3b2):
    full = lambda r, c: pl.BlockSpec((r, c), lambda i: (0, 0))
    blk = lambda c: pl.BlockSpec((BN, c), lambda i: (i, 0))
    return pl.pallas_call(
        _final_body,
        grid=(N // BN,),
        in_specs=[
            blk(DIM), blk(DIM), blk(DIM), blk(DIM), blk(16),
            full(DIM, DIM), full(1, DIM),
            full(DIM, DIM), full(1, DIM), full(DIM, 3), full(1, 3),
            full(DIM, DIM), full(1, DIM), full(DIM, 1), full(1, 1),
            full(DIM, DIM), full(DIM, DIM), full(1, DIM), full(DIM, DIM),
            full(1, DIM),
        ],
        out_specs=[blk(3), blk(DIM), blk(DIM)],
        out_shape=[
            jax.ShapeDtypeStruct((N, 3), jnp.float32),
            jax.ShapeDtypeStruct((N, DIM), jnp.float32),
            jax.ShapeDtypeStruct((N, DIM), jnp.float32),
        ],
    )(xh2, s0, s1, s2, sd, w2, b2, n1w1, n1b1, n1w2, n1b2,
      n2w1, n2b1, n2w2, n2b2, n3w1a, n3w1b, n3b1, n3w2, n3b2)


def kernel(x, xh, e, sc_W1, sc_b1, sc_W2, sc_b2, n1_W1, n1_b1, n1_W2, n1_b2,
           n2_W1, n2_b1, n2_W2, n2_b2, n3_W1, n3_b1, n3_W2, n3_b2):
    x2 = x[0]
    xh2 = xh[0]
    w0 = sc_W1[0:1]
    w1a = sc_W1[1:1+DIM]
    w1b = sc_W1[1+DIM:]
    b1 = sc_b1.reshape(1, DIM)

    a, b = _stage_ab(xh2, w1a, w1b, b1)

    xflat = jnp.pad(x2.reshape(3 * N), (0, XT_R * 128 - 3 * N)).reshape(XT_R, 128)
    pad = EP - E
    idx0 = e[0]
    idx1 = e[1]
    idx0g = jnp.pad(idx0, (0, pad)).reshape(1, EP)
    idx1g = jnp.pad(idx1, (0, pad)).reshape(1, EP)
    idx0p = jnp.pad(idx0, (0, pad), constant_values=-1)
    idx0s = idx0p.reshape(EP // 128, 128)
    idxf = jnp.pad(idx0p.astype(jnp.float32)[:, None], ((0, 0), (0, 15)))

    z, rd = _stage_gather(a, b, xflat, idx0g, idx1g)
    p0, p1, p2, pd, ie, ied = _stage_edge(z, rd, w0, idx0s, idxf)
    s0, s1, s2, sd2 = _stage_scatter(p0, p1, p2, pd, ie, ied)
    sd = jnp.concatenate(
        [sd2[c * ACCD_USED:(c + 1) * ACCD_USED].reshape(8 * ACCD_USED, 16)[:NHALF]
         for c in range(2)], axis=0)

    xo, nh, nrm = _stage_final(
        xh2, s0, s1, s2, sd, sc_W2, sc_b2.reshape(1, DIM),
        n1_W1, n1_b1.reshape(1, DIM), n1_W2, n1_b2.reshape(1, 3),
        n2_W1, n2_b1.reshape(1, DIM), n2_W2, n2_b2.reshape(1, 1),
        n3_W1[:DIM], n3_W1[DIM:], n3_b1.reshape(1, DIM),
        n3_W2, n3_b2.reshape(1, DIM))

    return (xo[None], nh[None], nrm[None])


# speedup vs baseline: 26.1417x; 1.0580x over previous
import dataclasses
import functools

import jax
import jax.numpy as jnp
from jax import lax
from jax.experimental import pallas as pl
from jax.experimental.pallas import tpu as pltpu
from jax.experimental.pallas import tpu_sc as plsc

N = 10000
E = 160000
DIM = 128
EP = 163840
GW = 128
BN = 1024
BE = 1024
NHALF = N // 2
N2 = 10240
ACC_R = 5120
TRASH = 5100
CH = 128


def _silu(v):
    return v * jax.nn.sigmoid(v)


def _sc_compiler_params():
    cp = pltpu.CompilerParams()
    if "needs_layout_passes" in pltpu.CompilerParams.__dataclass_fields__:
        cp = dataclasses.replace(cp, needs_layout_passes=False)
    return cp


def _ab_body(xh_ref, w1a_ref, w1b_ref, b1_ref, a_ref, b_ref):
    xh = xh_ref[...]
    a_ref[...] = jnp.dot(xh, w1a_ref[...], preferred_element_type=jnp.float32)
    b_ref[...] = (jnp.dot(xh, w1b_ref[...], preferred_element_type=jnp.float32)
                  + b1_ref[...])


def _stage_ab(xh2p, w1a, w1b, b1):
    bt = 1024
    return pl.pallas_call(
        _ab_body,
        grid=(N2 // bt,),
        in_specs=[
            pl.BlockSpec((bt, DIM), lambda i: (i, 0)),
            pl.BlockSpec((DIM, DIM), lambda i: (0, 0)),
            pl.BlockSpec((DIM, DIM), lambda i: (0, 0)),
            pl.BlockSpec((1, DIM), lambda i: (0, 0)),
        ],
        out_specs=[
            pl.BlockSpec((bt, DIM), lambda i: (i, 0)),
            pl.BlockSpec((bt, DIM), lambda i: (i, 0)),
        ],
        out_shape=[
            jax.ShapeDtypeStruct((N2, DIM), jnp.float32),
            jax.ShapeDtypeStruct((N2, DIM), jnp.float32),
        ],
    )(xh2p, w1a, w1b, b1)


XT_R = 235


def _stage_gather(a, b, xflat, idx0, idx1):
    mesh = plsc.VectorSubcoreMesh(core_axis_name="core",
                                  subcore_axis_name="subcore")

    @functools.partial(
        pl.kernel,
        out_type=[
            jax.ShapeDtypeStruct((EP, DIM), jnp.float32),
            jax.ShapeDtypeStruct((EP, 16), jnp.float32),
        ],
        mesh=mesh,
        scratch_types=[
            pltpu.VMEM((GW // 2, DIM), jnp.float32),
            pltpu.VMEM((GW // 2, DIM), jnp.float32),
            pltpu.VMEM((XT_R, 128), jnp.float32),
            pltpu.SemaphoreType.DMA,
            pltpu.SemaphoreType.DMA,
        ],
        compiler_params=_sc_compiler_params(),
    )
    def k(a_hbm, b_hbm, x_hbm, i0_hbm, i1_hbm, z_hbm, rd_hbm,
          ga, gb, xt, sga, sgb):
        pltpu.sync_copy(x_hbm, xt)

        def body(i0_v, i1_v, z_v, rd_v):
            for h in range(2):
                hs = h * (GW // 2)
                ca = pltpu.async_copy(
                    a_hbm.at[i0_v.at[0, pl.ds(hs, GW // 2)]], ga, sga)
                cb = pltpu.async_copy(
                    b_hbm.at[i1_v.at[0, pl.ds(hs, GW // 2)]], gb, sgb)

                @pl.loop(0, GW // 32)
                def _(g):
                    e0 = hs + g * 16
                    i0 = i0_v[0, pl.ds(e0, 16)]
                    i1 = i1_v[0, pl.ds(e0, 16)]
                    rows = lax.iota(jnp.int32, 16) + e0
                    d = []
                    for c in range(3):
                        f0 = i0 * 3 + c
                        f1 = i1 * 3 + c
                        v0 = plsc.load_gather(xt, [f0 >> 7, f0 & 127])
                        v1 = plsc.load_gather(xt, [f1 >> 7, f1 & 127])
                        d.append(v0 - v1)
                    qsq = d[0] * d[0] + d[1] * d[1] + d[2] * d[2]
                    i32 = jnp.int32(0x5f3759df) - (plsc.bitcast(qsq, jnp.int32) >> 1)
                    y = plsc.bitcast(i32, jnp.float32)
                    for _ in range(3):
                        y = y * (1.5 - 0.5 * qsq * y * y)
                    for c in range(3):
                        dn = d[c] * y
                        plsc.store_scatter(
                            rd_v, [rows, jnp.full((16,), c, jnp.int32)], dn)
                    plsc.store_scatter(
                        rd_v, [rows, jnp.full((16,), 3, jnp.int32)], qsq)

                ca.wait()
                cb.wait()

                @pl.loop(0, GW // 2)
                def _(r):
                    for c in range(DIM // 16):
                        sl = pl.ds(c * 16, 16)
                        z_v[hs + r, sl] = ga[r, sl] + gb[r, sl]

        pltpu.emit_pipeline(
            body,
            grid=(EP // GW,),
            in_specs=[
                pl.BlockSpec((1, GW), lambda i: (0, i)),
                pl.BlockSpec((1, GW), lambda i: (0, i)),
            ],
            out_specs=[
                pl.BlockSpec((GW, DIM), lambda i: (i, 0)),
                pl.BlockSpec((GW, 16), lambda i: (i, 0)),
            ],
            core_axis_name=("core", "subcore"),
            dimension_semantics=(pltpu.PARALLEL,),
        )(i0_hbm, i1_hbm, z_hbm, rd_hbm)

    return k(a, b, xflat, idx0, idx1)


def _edge_body(z_ref, rd_ref, w0_ref, idx_ref, p0_ref, p1_ref, p2_ref,
               ie_ref):
    z = z_ref[...]
    rd = rd_ref[...]
    d = rd[:, 0:3]
    qsq = rd[:, 3:4]
    zz = z + qsq * w0_ref[...]
    h = _silu(zz)
    p0_ref[...] = h * d[:, 0:1]
    p1_ref[...] = h * d[:, 1:2]
    p2_ref[...] = h * d[:, 2:3]
    idx = idx_ref[...]
    in0 = (idx >= 0) & (idx < NHALF)
    in1 = (idx >= NHALF) & (idx < N)
    ie_ref[0] = jnp.where(in0, idx, TRASH)
    ie_ref[1] = jnp.where(in1, idx - NHALF, TRASH)


def _stage_edge(z, rd, w0, idx0s):
    ieb = pl.BlockSpec((2, BE // 128, 128), lambda i: (0, i, 0))
    ies = jax.ShapeDtypeStruct((2, EP // 128, 128), jnp.int32)
    return pl.pallas_call(
        _edge_body,
        grid=(EP // BE,),
        in_specs=[
            pl.BlockSpec((BE, DIM), lambda i: (i, 0)),
            pl.BlockSpec((BE, 16), lambda i: (i, 0)),
            pl.BlockSpec((1, DIM), lambda i: (0, 0)),
            pl.BlockSpec((BE // 128, 128), lambda i: (i, 0)),
        ],
        out_specs=[
            pl.BlockSpec((BE, DIM), lambda i: (i, 0)),
            pl.BlockSpec((BE, DIM), lambda i: (i, 0)),
            pl.BlockSpec((BE, DIM), lambda i: (i, 0)),
            ieb,
        ],
        out_shape=[
            jax.ShapeDtypeStruct((EP, DIM), jnp.float32),
            jax.ShapeDtypeStruct((EP, DIM), jnp.float32),
            jax.ShapeDtypeStruct((EP, DIM), jnp.float32),
            ies,
        ],
    )(z, rd, w0, idx0s)


def _stage_scatter(p0, p1, p2, rd, ie):
    mesh = plsc.VectorSubcoreMesh(core_axis_name="core",
                                  subcore_axis_name="subcore")
    n_chunks = EP // (16 * CH)

    @functools.partial(
        pl.kernel,
        out_type=[
            jax.ShapeDtypeStruct((N2, DIM), jnp.float32),
            jax.ShapeDtypeStruct((N2, DIM), jnp.float32),
            jax.ShapeDtypeStruct((N2, DIM), jnp.float32),
            jax.ShapeDtypeStruct((2, 64, ACC_R), jnp.float32),
        ],
        mesh=mesh,
        scratch_types=[
            pltpu.VMEM((CH, DIM), jnp.float32),
            pltpu.VMEM((CH, DIM), jnp.float32),
            pltpu.VMEM((CH, 16), jnp.float32),
            pltpu.VMEM((1, CH), jnp.int32),
            pltpu.VMEM((1, CH), jnp.int32),
            pltpu.VMEM((4, ACC_R), jnp.float32),
            pltpu.VMEM((16, DIM), jnp.float32),
            pltpu.VMEM((40, DIM), jnp.float32),
            pltpu.VMEM_SHARED((ACC_R, DIM), jnp.float32),
            pltpu.SemaphoreType.DMA,
            pltpu.SemaphoreType.DMA,
        ],
        compiler_params=_sc_compiler_params(),
    )
    def k(p0_h, p1_h, p2_h, rd_h, ie_h, s0_h, s1_h, s2_h, dsp_h,
          pba, pbb, rdb, iba, ibb, priv, zb, db, acc, sema, semb):
        c = lax.axis_index("core")
        s = lax.axis_index("subcore")

        @pl.loop(0, 16)
        def _(r):
            for cc in range(DIM // 16):
                zb[r, pl.ds(cc * 16, 16)] = jnp.zeros((16,), jnp.float32)

        @pl.loop(0, 4)
        def _(r):
            @pl.loop(0, ACC_R // 16)
            def _(j):
                priv[r, pl.ds(j * 16, 16)] = jnp.zeros((16,), jnp.float32)

        for kpass, (p_h, s_h) in enumerate(
                ((p0_h, s0_h), (p1_h, s1_h), (p2_h, s2_h))):

            @pl.loop(0, 20)
            def _(t):
                ro = s * 320 + t * 16
                pltpu.sync_copy(zb, acc.at[pl.ds(ro, 16)])

            plsc.subcore_barrier()

            def issue(t, pb_, ib_, sem):
                base = (s * n_chunks + t) * CH
                pltpu.async_copy(ie_h.at[c, pl.ds(base, CH)], ib_.at[0], sem)
                pltpu.async_copy(p_h.at[pl.ds(base, CH)], pb_, sem)

            def drain(t, pb_, ib_, sem):
                base = (s * n_chunks + t) * CH
                pltpu.make_async_copy(ie_h.at[c, pl.ds(base, CH)], ib_.at[0],
                                      sem).wait()
                pltpu.make_async_copy(p_h.at[pl.ds(base, CH)], pb_, sem).wait()

            def accum_d(t, ib_):
                if kpass != 0:
                    return
                base = (s * n_chunks + t) * CH
                pltpu.sync_copy(rd_h.at[pl.ds(base, CH)], rdb)

                @pl.loop(0, CH // 16)
                def _(g):
                    iloc = ib_[0, pl.ds(g * 16, 16)]
                    rows = lax.iota(jnp.int32, 16) + g * 16
                    for cc in range(3):
                        dc = plsc.load_gather(
                            rdb, [rows, jnp.full((16,), cc, jnp.int32)])
                        plsc.addupdate_scatter(
                            priv, [jnp.full((16,), cc, jnp.int32), iloc], dc)
                    plsc.addupdate_scatter(
                        priv, [jnp.full((16,), 3, jnp.int32), iloc],
                        jnp.full((16,), 1.0, jnp.float32))

            issue(0, pba, iba, sema)

            @pl.loop(0, n_chunks, step=2)
            def _(t):
                issue(t + 1, pbb, ibb, semb)
                drain(t, pba, iba, sema)
                pltpu.sync_copy(pba, acc.at[iba.at[0]], add=True)
                accum_d(t, iba)

                @pl.when(t + 2 < n_chunks)
                def _():
                    issue(t + 2, pba, iba, sema)

                drain(t + 1, pbb, ibb, semb)
                pltpu.sync_copy(pbb, acc.at[ibb.at[0]], add=True)
                accum_d(t + 1, ibb)

            plsc.subcore_barrier()

            @pl.loop(0, 8)
            def _(t):
                cidx = s + t * 16

                @pl.when(cidx < NHALF // 40)
                def _():
                    ro = cidx * 40
                    pltpu.sync_copy(acc.at[pl.ds(ro, 40)], db)
                    pltpu.sync_copy(db, s_h.at[pl.ds(c * NHALF + ro, 40)])

            plsc.subcore_barrier()

        pltpu.sync_copy(priv, dsp_h.at[c, pl.ds(s * 4, 4)])

    ief = ie.reshape(2, EP)
    return k(p0, p1, p2, rd, ief)


def _final_body(xh_ref, s0_ref, s1_ref, s2_ref, dsp_ref, sel_ref, w2_ref,
                b2_ref,
                n1w1_ref, n1b1_ref, n1w2_ref, n1b2_ref,
                n2w1_ref, n2b1_ref, n2w2_ref, n2b2_ref,
                n3w1a_ref, n3w1b_ref, n3b1_ref, n3w2_ref, n3b2_ref,
                xo_ref, nh_ref, nrm_ref):
    xh = xh_ref[...]
    sd = lax.dot_general(dsp_ref[...], sel_ref[...],
                         (((0,), (0,)), ((), ())),
                         preferred_element_type=jnp.float32)
    deg = sd[:, 3:4]
    inv = 1.0 / jnp.maximum(deg, 1.0)
    w2 = w2_ref[...]
    b2 = b2_ref[...]
    means = []
    for k, s_ref in enumerate((s0_ref, s1_ref, s2_ref)):
        sk = s_ref[...]
        mk = (jnp.dot(sk, w2, preferred_element_type=jnp.float32)
              + b2 * sd[:, k:k+1]) * inv
        means.append(mk)
    nrm = jnp.sqrt(means[0] * means[0] + means[1] * means[1]
                   + means[2] * means[2])
    nrm_ref[...] = nrm
    t1 = _silu(jnp.dot(xh, n1w1_ref[...], preferred_element_type=jnp.float32)
               + n1b1_ref[...])
    o1 = jnp.dot(t1, n1w2_ref[...], preferred_element_type=jnp.float32) + n1b2_ref[...]
    cols = []
    for k in range(3):
        u = _silu(jnp.dot(means[k], n2w1_ref[...],
                          preferred_element_type=jnp.float32) + n2b1_ref[...])
        cols.append(jnp.dot(u, n2w2_ref[...],
                            preferred_element_type=jnp.float32) + n2b2_ref[...])
    xo_ref[...] = o1 + jnp.concatenate(cols, axis=1)
    g = _silu(jnp.dot(xh, n3w1a_ref[...], preferred_element_type=jnp.float32)
              + jnp.dot(nrm, n3w1b_ref[...], preferred_element_type=jnp.float32)
              + n3b1_ref[...])
    nh_ref[...] = (jnp.dot(g, n3w2_ref[...], preferred_element_type=jnp.float32)
                   + n3b2_ref[...])


def _stage_final(xh2, s0, s1, s2, dsp, sel, w2, b2, n1w1, n1b1, n1w2, n1b2,
                 n2w1, n2b1, n2w2, n2b2, n3w1a, n3w1b, n3b1, n3w2, n3b2):
    full = lambda r, c: pl.BlockSpec((r, c), lambda i: (0, 0))
    blk = lambda c: pl.BlockSpec((BN, c), lambda i: (i, 0))
    return pl.pallas_call(
        _final_body,
        grid=(N2 // BN,),
        in_specs=[
            blk(DIM), blk(DIM), blk(DIM), blk(DIM),
            pl.BlockSpec((64, BN), lambda i: (0, i)),
            full(64, 4),
            full(DIM, DIM), full(1, DIM),
            full(DIM, DIM), full(1, DIM), full(DIM, 3), full(1, 3),
            full(DIM, DIM), full(1, DIM), full(DIM, 1), full(1, 1),
            full(DIM, DIM), full(DIM, DIM), full(1, DIM), full(DIM, DIM),
            full(1, DIM),
        ],
        out_specs=[blk(3), blk(DIM), blk(DIM)],
        out_shape=[
            jax.ShapeDtypeStruct((N2, 3), jnp.float32),
            jax.ShapeDtypeStruct((N2, DIM), jnp.float32),
            jax.ShapeDtypeStruct((N2, DIM), jnp.float32),
        ],
    )(xh2, s0, s1, s2, dsp, sel, w2, b2, n1w1, n1b1, n1w2, n1b2,
      n2w1, n2b1, n2w2, n2b2, n3w1a, n3w1b, n3b1, n3w2, n3b2)


def kernel(x, xh, e, sc_W1, sc_b1, sc_W2, sc_b2, n1_W1, n1_b1, n1_W2, n1_b2,
           n2_W1, n2_b1, n2_W2, n2_b2, n3_W1, n3_b1, n3_W2, n3_b2):
    x2 = x[0]
    xh2 = xh[0]
    w0 = sc_W1[0:1]
    w1a = sc_W1[1:1+DIM]
    w1b = sc_W1[1+DIM:]
    b1 = sc_b1.reshape(1, DIM)

    xh2p = jnp.pad(xh2, ((0, N2 - N), (0, 0)))
    a, b = _stage_ab(xh2p, w1a, w1b, b1)

    xflat = jnp.pad(x2.reshape(3 * N), (0, XT_R * 128 - 3 * N)).reshape(XT_R, 128)
    pad = EP - E
    idx0 = e[0]
    idx1 = e[1]
    idx0g = jnp.pad(idx0, (0, pad), constant_values=N).reshape(1, EP)
    idx1g = jnp.pad(idx1, (0, pad), constant_values=N).reshape(1, EP)
    idx0s = jnp.pad(idx0, (0, pad), constant_values=-1).reshape(EP // 128, 128)
    sel = jnp.tile(jnp.eye(4, dtype=jnp.float32), (16, 1))

    z, rd = _stage_gather(a, b, xflat, idx0g, idx1g)
    p0, p1, p2, ie = _stage_edge(z, rd, w0, idx0s)
    s0, s1, s2, dsp = _stage_scatter(p0, p1, p2, rd, ie)
    dspp = jnp.pad(
        jnp.concatenate([dsp[0, :, :NHALF], dsp[1, :, :NHALF]], axis=1),
        ((0, 0), (0, N2 - N)))

    xo, nh, nrm = _stage_final(
        xh2p, s0, s1, s2, dspp, sel, sc_W2, sc_b2.reshape(1, DIM),
        n1_W1, n1_b1.reshape(1, DIM), n1_W2, n1_b2.reshape(1, 3),
        n2_W1, n2_b1.reshape(1, DIM), n2_W2, n2_b2.reshape(1, 1),
        n3_W1[:DIM], n3_W1[DIM:], n3_b1.reshape(1, DIM),
        n3_W2, n3_b2.reshape(1, DIM))

    return (xo[:N][None], nh[:N][None], nrm[:N][None])


# speedup vs baseline: 27.5474x; 1.0538x over previous
import dataclasses
import functools

import jax
import jax.numpy as jnp
from jax import lax
from jax.experimental import pallas as pl
from jax.experimental.pallas import tpu as pltpu
from jax.experimental.pallas import tpu_sc as plsc

N = 10000
E = 160000
DIM = 128
EP = 163840
GW = 128
BN = 1024
BE = 4096
NHALF = N // 2
N2 = 10240
ACC_R = 5120
TRASH = 5100
CH = 128


def _silu(v):
    return v * jax.nn.sigmoid(v)


def _sc_compiler_params():
    cp = pltpu.CompilerParams()
    if "needs_layout_passes" in pltpu.CompilerParams.__dataclass_fields__:
        cp = dataclasses.replace(cp, needs_layout_passes=False)
    return cp


def _ab_body(xh_ref, w1a_ref, w1b_ref, b1_ref, a_ref, b_ref):
    xh = xh_ref[...]
    a_ref[...] = jnp.dot(xh, w1a_ref[...], preferred_element_type=jnp.float32)
    b_ref[...] = (jnp.dot(xh, w1b_ref[...], preferred_element_type=jnp.float32)
                  + b1_ref[...])


def _stage_ab(xh2p, w1a, w1b, b1):
    bt = 1024
    return pl.pallas_call(
        _ab_body,
        grid=(N2 // bt,),
        in_specs=[
            pl.BlockSpec((bt, DIM), lambda i: (i, 0)),
            pl.BlockSpec((DIM, DIM), lambda i: (0, 0)),
            pl.BlockSpec((DIM, DIM), lambda i: (0, 0)),
            pl.BlockSpec((1, DIM), lambda i: (0, 0)),
        ],
        out_specs=[
            pl.BlockSpec((bt, DIM), lambda i: (i, 0)),
            pl.BlockSpec((bt, DIM), lambda i: (i, 0)),
        ],
        out_shape=[
            jax.ShapeDtypeStruct((N2, DIM), jnp.float32),
            jax.ShapeDtypeStruct((N2, DIM), jnp.float32),
        ],
    )(xh2p, w1a, w1b, b1)


XT_R = 235


def _stage_gather(a, b, xflat, idx0, idx1):
    mesh = plsc.VectorSubcoreMesh(core_axis_name="core",
                                  subcore_axis_name="subcore")

    @functools.partial(
        pl.kernel,
        out_type=[
            jax.ShapeDtypeStruct((EP, DIM), jnp.float32),
            jax.ShapeDtypeStruct((EP, 16), jnp.float32),
        ],
        mesh=mesh,
        scratch_types=[
            pltpu.VMEM((GW // 2, DIM), jnp.float32),
            pltpu.VMEM((GW // 2, DIM), jnp.float32),
            pltpu.VMEM((XT_R, 128), jnp.float32),
            pltpu.SemaphoreType.DMA,
            pltpu.SemaphoreType.DMA,
        ],
        compiler_params=_sc_compiler_params(),
    )
    def k(a_hbm, b_hbm, x_hbm, i0_hbm, i1_hbm, z_hbm, rd_hbm,
          ga, gb, xt, sga, sgb):
        pltpu.sync_copy(x_hbm, xt)

        def body(i0_v, i1_v, z_v, rd_v):
            for h in range(2):
                hs = h * (GW // 2)
                ca = pltpu.async_copy(
                    a_hbm.at[i0_v.at[0, pl.ds(hs, GW // 2)]], ga, sga)
                cb = pltpu.async_copy(
                    b_hbm.at[i1_v.at[0, pl.ds(hs, GW // 2)]], gb, sgb)

                @pl.loop(0, GW // 32)
                def _(g):
                    e0 = hs + g * 16
                    i0 = i0_v[0, pl.ds(e0, 16)]
                    i1 = i1_v[0, pl.ds(e0, 16)]
                    rows = lax.iota(jnp.int32, 16) + e0
                    d = []
                    for c in range(3):
                        f0 = i0 * 3 + c
                        f1 = i1 * 3 + c
                        v0 = plsc.load_gather(xt, [f0 >> 7, f0 & 127])
                        v1 = plsc.load_gather(xt, [f1 >> 7, f1 & 127])
                        d.append(v0 - v1)
                    qsq = d[0] * d[0] + d[1] * d[1] + d[2] * d[2]
                    i32 = jnp.int32(0x5f3759df) - (plsc.bitcast(qsq, jnp.int32) >> 1)
                    y = plsc.bitcast(i32, jnp.float32)
                    for _ in range(3):
                        y = y * (1.5 - 0.5 * qsq * y * y)
                    for c in range(3):
                        dn = d[c] * y
                        plsc.store_scatter(
                            rd_v, [rows, jnp.full((16,), c, jnp.int32)], dn)
                    plsc.store_scatter(
                        rd_v, [rows, jnp.full((16,), 3, jnp.int32)], qsq)

                ca.wait()
                cb.wait()

                @pl.loop(0, GW // 2)
                def _(r):
                    for c in range(DIM // 16):
                        sl = pl.ds(c * 16, 16)
                        z_v[hs + r, sl] = ga[r, sl] + gb[r, sl]

        pltpu.emit_pipeline(
            body,
            grid=(EP // GW,),
            in_specs=[
                pl.BlockSpec((1, GW), lambda i: (0, i)),
                pl.BlockSpec((1, GW), lambda i: (0, i)),
            ],
            out_specs=[
                pl.BlockSpec((GW, DIM), lambda i: (i, 0)),
                pl.BlockSpec((GW, 16), lambda i: (i, 0)),
            ],
            core_axis_name=("core", "subcore"),
            dimension_semantics=(pltpu.PARALLEL,),
        )(i0_hbm, i1_hbm, z_hbm, rd_hbm)

    return k(a, b, xflat, idx0, idx1)


def _edge_body(z_ref, rd_ref, w0_ref, idx_ref, p0_ref, p1_ref, p2_ref,
               ie_ref):
    z = z_ref[...]
    rd = rd_ref[...]
    d = rd[:, 0:3]
    qsq = rd[:, 3:4]
    zz = z + qsq * w0_ref[...]
    h = _silu(zz)
    p0_ref[...] = h * d[:, 0:1]
    p1_ref[...] = h * d[:, 1:2]
    p2_ref[...] = h * d[:, 2:3]
    idx = idx_ref[...]
    in0 = (idx >= 0) & (idx < NHALF)
    in1 = (idx >= NHALF) & (idx < N)
    ie_ref[0] = jnp.where(in0, idx, TRASH)
    ie_ref[1] = jnp.where(in1, idx - NHALF, TRASH)


def _stage_edge(z, rd, w0, idx0s):
    ieb = pl.BlockSpec((2, BE // 128, 128), lambda i: (0, i, 0))
    ies = jax.ShapeDtypeStruct((2, EP // 128, 128), jnp.int32)
    return pl.pallas_call(
        _edge_body,
        grid=(EP // BE,),
        in_specs=[
            pl.BlockSpec((BE, DIM), lambda i: (i, 0)),
            pl.BlockSpec((BE, 16), lambda i: (i, 0)),
            pl.BlockSpec((1, DIM), lambda i: (0, 0)),
            pl.BlockSpec((BE // 128, 128), lambda i: (i, 0)),
        ],
        out_specs=[
            pl.BlockSpec((BE, DIM), lambda i: (i, 0)),
            pl.BlockSpec((BE, DIM), lambda i: (i, 0)),
            pl.BlockSpec((BE, DIM), lambda i: (i, 0)),
            ieb,
        ],
        out_shape=[
            jax.ShapeDtypeStruct((EP, DIM), jnp.float32),
            jax.ShapeDtypeStruct((EP, DIM), jnp.float32),
            jax.ShapeDtypeStruct((EP, DIM), jnp.float32),
            ies,
        ],
    )(z, rd, w0, idx0s)


def _stage_scatter(p0, p1, p2, rd, ie):
    mesh = plsc.VectorSubcoreMesh(core_axis_name="core",
                                  subcore_axis_name="subcore")
    ck = CH
    n_chunks = EP // (16 * ck)

    @functools.partial(
        pl.kernel,
        out_type=[
            jax.ShapeDtypeStruct((N2, DIM), jnp.float32),
            jax.ShapeDtypeStruct((N2, DIM), jnp.float32),
            jax.ShapeDtypeStruct((N2, DIM), jnp.float32),
            jax.ShapeDtypeStruct((2, 64, ACC_R), jnp.float32),
        ],
        mesh=mesh,
        scratch_types=[
            pltpu.VMEM((CH, DIM), jnp.float32),
            pltpu.VMEM((CH, DIM), jnp.float32),
            pltpu.VMEM((CH, 16), jnp.float32),
            pltpu.VMEM((1, CH), jnp.int32),
            pltpu.VMEM((1, CH), jnp.int32),
            pltpu.VMEM((4, ACC_R), jnp.float32),
            pltpu.VMEM((16, DIM), jnp.float32),
            pltpu.VMEM((40, DIM), jnp.float32),
            pltpu.VMEM_SHARED((ACC_R, DIM), jnp.float32),
            pltpu.SemaphoreType.DMA,
            pltpu.SemaphoreType.DMA,
        ],
        compiler_params=_sc_compiler_params(),
    )
    def k(p0_h, p1_h, p2_h, rd_h, ie_h, s0_h, s1_h, s2_h, dsp_h,
          pba, pbb, rdb, iba, ibb, priv, zb, db, acc, sema, semb):
        c = lax.axis_index("core")
        s = lax.axis_index("subcore")

        @pl.loop(0, 16)
        def _(r):
            for cc in range(DIM // 16):
                zb[r, pl.ds(cc * 16, 16)] = jnp.zeros((16,), jnp.float32)

        @pl.loop(0, 4)
        def _(r):
            @pl.loop(0, ACC_R // 16)
            def _(j):
                priv[r, pl.ds(j * 16, 16)] = jnp.zeros((16,), jnp.float32)

        for kpass, (p_h, s_h) in enumerate(
                ((p0_h, s0_h), (p1_h, s1_h), (p2_h, s2_h))):

            @pl.loop(0, 20)
            def _(t):
                ro = s * 320 + t * 16
                pltpu.sync_copy(zb, acc.at[pl.ds(ro, 16)])

            plsc.subcore_barrier()

            def issue(t, pb_, ib_, sem):
                base = (s * n_chunks + t) * ck
                pltpu.async_copy(ie_h.at[c, pl.ds(base, CH)], ib_.at[0], sem)
                pltpu.async_copy(p_h.at[pl.ds(base, ck)], pb_, sem)

            def drain(t, pb_, ib_, sem):
                base = (s * n_chunks + t) * ck
                pltpu.make_async_copy(ie_h.at[c, pl.ds(base, CH)], ib_.at[0],
                                      sem).wait()
                pltpu.make_async_copy(p_h.at[pl.ds(base, ck)], pb_, sem).wait()

            def scat(pb_, ib_):
                pltpu.sync_copy(pb_, acc.at[ib_.at[0]], add=True)

            def accum_d(t, ib_):
                if kpass != 0:
                    return
                base = (s * n_chunks + t) * ck
                pltpu.sync_copy(rd_h.at[pl.ds(base, ck)], rdb)

                @pl.loop(0, ck // 16)
                def _(g):
                    iloc = ib_[0, pl.ds(g * 16, 16)]
                    rows = lax.iota(jnp.int32, 16) + g * 16
                    for cc in range(3):
                        dc = plsc.load_gather(
                            rdb, [rows, jnp.full((16,), cc, jnp.int32)])
                        plsc.addupdate_scatter(
                            priv, [jnp.full((16,), cc, jnp.int32), iloc], dc)
                    plsc.addupdate_scatter(
                        priv, [jnp.full((16,), 3, jnp.int32), iloc],
                        jnp.full((16,), 1.0, jnp.float32))

            issue(0, pba, iba, sema)

            @pl.loop(0, n_chunks, step=2)
            def _(t):
                issue(t + 1, pbb, ibb, semb)
                drain(t, pba, iba, sema)
                scat(pba, iba)
                accum_d(t, iba)

                @pl.when(t + 2 < n_chunks)
                def _():
                    issue(t + 2, pba, iba, sema)

                drain(t + 1, pbb, ibb, semb)
                scat(pbb, ibb)
                accum_d(t + 1, ibb)

            plsc.subcore_barrier()

            @pl.loop(0, 8)
            def _(t):
                cidx = s + t * 16

                @pl.when(cidx < NHALF // 40)
                def _():
                    ro = cidx * 40
                    pltpu.sync_copy(acc.at[pl.ds(ro, 40)], db)
                    pltpu.sync_copy(db, s_h.at[pl.ds(c * NHALF + ro, 40)])

            plsc.subcore_barrier()

        pltpu.sync_copy(priv, dsp_h.at[c, pl.ds(s * 4, 4)])

    return k(p0, p1, p2, rd, ie.reshape(2, EP))


def _final_body(xh_ref, s0_ref, s1_ref, s2_ref, dsp_ref, sel_ref, w2_ref,
                b2_ref,
                n1w1_ref, n1b1_ref, n1w2_ref, n1b2_ref,
                n2w1_ref, n2b1_ref, n2w2_ref, n2b2_ref,
                n3w1a_ref, n3w1b_ref, n3b1_ref, n3w2_ref, n3b2_ref,
                xo_ref, nh_ref, nrm_ref):
    xh = xh_ref[...]
    sd = lax.dot_general(dsp_ref[...], sel_ref[...],
                         (((0,), (0,)), ((), ())),
                         preferred_element_type=jnp.float32)
    deg = sd[:, 3:4]
    inv = 1.0 / jnp.maximum(deg, 1.0)
    w2 = w2_ref[...]
    b2 = b2_ref[...]
    means = []
    for k, s_ref in enumerate((s0_ref, s1_ref, s2_ref)):
        sk = s_ref[...]
        mk = (jnp.dot(sk, w2, preferred_element_type=jnp.float32)
              + b2 * sd[:, k:k+1]) * inv
        means.append(mk)
    nrm = jnp.sqrt(means[0] * means[0] + means[1] * means[1]
                   + means[2] * means[2])
    nrm_ref[...] = nrm
    t1 = _silu(jnp.dot(xh, n1w1_ref[...], preferred_element_type=jnp.float32)
               + n1b1_ref[...])
    o1 = jnp.dot(t1, n1w2_ref[...], preferred_element_type=jnp.float32) + n1b2_ref[...]
    cols = []
    for k in range(3):
        u = _silu(jnp.dot(means[k], n2w1_ref[...],
                          preferred_element_type=jnp.float32) + n2b1_ref[...])
        cols.append(jnp.dot(u, n2w2_ref[...],
                            preferred_element_type=jnp.float32) + n2b2_ref[...])
    xo_ref[...] = o1 + jnp.concatenate(cols, axis=1)
    g = _silu(jnp.dot(xh, n3w1a_ref[...], preferred_element_type=jnp.float32)
              + jnp.dot(nrm, n3w1b_ref[...], preferred_element_type=jnp.float32)
              + n3b1_ref[...])
    nh_ref[...] = (jnp.dot(g, n3w2_ref[...], preferred_element_type=jnp.float32)
                   + n3b2_ref[...])


def _stage_final(xh2, s0, s1, s2, dsp, sel, w2, b2, n1w1, n1b1, n1w2, n1b2,
                 n2w1, n2b1, n2w2, n2b2, n3w1a, n3w1b, n3b1, n3w2, n3b2):
    full = lambda r, c: pl.BlockSpec((r, c), lambda i: (0, 0))
    blk = lambda c: pl.BlockSpec((BN, c), lambda i: (i, 0))
    return pl.pallas_call(
        _final_body,
        grid=(N2 // BN,),
        in_specs=[
            blk(DIM), blk(DIM), blk(DIM), blk(DIM),
            pl.BlockSpec((64, BN), lambda i: (0, i)),
            full(64, 4),
            full(DIM, DIM), full(1, DIM),
            full(DIM, DIM), full(1, DIM), full(DIM, 3), full(1, 3),
            full(DIM, DIM), full(1, DIM), full(DIM, 1), full(1, 1),
            full(DIM, DIM), full(DIM, DIM), full(1, DIM), full(DIM, DIM),
            full(1, DIM),
        ],
        out_specs=[blk(3), blk(DIM), blk(DIM)],
        out_shape=[
            jax.ShapeDtypeStruct((N2, 3), jnp.float32),
            jax.ShapeDtypeStruct((N2, DIM), jnp.float32),
            jax.ShapeDtypeStruct((N2, DIM), jnp.float32),
        ],
    )(xh2, s0, s1, s2, dsp, sel, w2, b2, n1w1, n1b1, n1w2, n1b2,
      n2w1, n2b1, n2w2, n2b2, n3w1a, n3w1b, n3b1, n3w2, n3b2)


def kernel(x, xh, e, sc_W1, sc_b1, sc_W2, sc_b2, n1_W1, n1_b1, n1_W2, n1_b2,
           n2_W1, n2_b1, n2_W2, n2_b2, n3_W1, n3_b1, n3_W2, n3_b2):
    x2 = x[0]
    xh2 = xh[0]
    w0 = sc_W1[0:1]
    w1a = sc_W1[1:1+DIM]
    w1b = sc_W1[1+DIM:]
    b1 = sc_b1.reshape(1, DIM)

    xh2p = jnp.pad(xh2, ((0, N2 - N), (0, 0)))
    a, b = _stage_ab(xh2p, w1a, w1b, b1)

    xflat = jnp.pad(x2.reshape(3 * N), (0, XT_R * 128 - 3 * N)).reshape(XT_R, 128)
    pad = EP - E
    idx0 = e[0]
    idx1 = e[1]
    idx0g = jnp.pad(idx0, (0, pad), constant_values=N).reshape(1, EP)
    idx1g = jnp.pad(idx1, (0, pad), constant_values=N).reshape(1, EP)
    idx0s = jnp.pad(idx0, (0, pad), constant_values=-1).reshape(EP // 128, 128)
    sel = jnp.tile(jnp.eye(4, dtype=jnp.float32), (16, 1))

    z, rd = _stage_gather(a, b, xflat, idx0g, idx1g)
    p0, p1, p2, ie = _stage_edge(z, rd, w0, idx0s)
    s0, s1, s2, dsp = _stage_scatter(p0, p1, p2, rd, ie)
    dspp = jnp.pad(
        jnp.concatenate([dsp[0, :, :NHALF], dsp[1, :, :NHALF]], axis=1),
        ((0, 0), (0, N2 - N)))

    xo, nh, nrm = _stage_final(
        xh2p, s0, s1, s2, dspp, sel, sc_W2, sc_b2.reshape(1, DIM),
        n1_W1, n1_b1.reshape(1, DIM), n1_W2, n1_b2.reshape(1, 3),
        n2_W1, n2_b1.reshape(1, DIM), n2_W2, n2_b2.reshape(1, 1),
        n3_W1[:DIM], n3_W1[DIM:], n3_b1.reshape(1, DIM),
        n3_W2, n3_b2.reshape(1, DIM))

    return (xo[:N][None], nh[:N][None], nrm[:N][None])


# speedup vs baseline: 29.8313x; 1.0829x over previous
import dataclasses
import functools

import jax
import jax.numpy as jnp
from jax import lax
from jax.experimental import pallas as pl
from jax.experimental.pallas import tpu as pltpu
from jax.experimental.pallas import tpu_sc as plsc

N = 10000
E = 160000
DIM = 128
EP = 163840
GW = 128
BN = 1024
BE = 4096
NHALF = N // 2
N2 = 10240
ACC_R = 5120
TRASH = 5100
CH = 128


def _silu(v):
    return v * jax.nn.sigmoid(v)


def _sc_compiler_params():
    cp = pltpu.CompilerParams()
    if "needs_layout_passes" in pltpu.CompilerParams.__dataclass_fields__:
        cp = dataclasses.replace(cp, needs_layout_passes=False)
    return cp


def _ab_body(xh_ref, w1a_ref, w1b_ref, b1_ref, a_ref, b_ref):
    xh = xh_ref[...]
    a_ref[...] = jnp.dot(xh, w1a_ref[...], preferred_element_type=jnp.float32)
    b_ref[...] = (jnp.dot(xh, w1b_ref[...], preferred_element_type=jnp.float32)
                  + b1_ref[...])


def _stage_ab(xh2p, w1a, w1b, b1):
    bt = 1024
    return pl.pallas_call(
        _ab_body,
        grid=(N2 // bt,),
        in_specs=[
            pl.BlockSpec((bt, DIM), lambda i: (i, 0)),
            pl.BlockSpec((DIM, DIM), lambda i: (0, 0)),
            pl.BlockSpec((DIM, DIM), lambda i: (0, 0)),
            pl.BlockSpec((1, DIM), lambda i: (0, 0)),
        ],
        out_specs=[
            pl.BlockSpec((bt, DIM), lambda i: (i, 0)),
            pl.BlockSpec((bt, DIM), lambda i: (i, 0)),
        ],
        out_shape=[
            jax.ShapeDtypeStruct((N2, DIM), jnp.float32),
            jax.ShapeDtypeStruct((N2, DIM), jnp.float32),
        ],
    )(xh2p, w1a, w1b, b1)


XT_R = 235


def _stage_gather(a, b, xflat, idx0, idx1):
    mesh = plsc.VectorSubcoreMesh(core_axis_name="core",
                                  subcore_axis_name="subcore")

    @functools.partial(
        pl.kernel,
        out_type=[
            jax.ShapeDtypeStruct((EP, DIM), jnp.float32),
            jax.ShapeDtypeStruct((EP, 16), jnp.float32),
        ],
        mesh=mesh,
        scratch_types=[
            pltpu.VMEM((GW // 2, DIM), jnp.float32),
            pltpu.VMEM((GW // 2, DIM), jnp.float32),
            pltpu.VMEM((GW // 2, DIM), jnp.float32),
            pltpu.VMEM((GW // 2, DIM), jnp.float32),
            pltpu.VMEM((XT_R, 128), jnp.float32),
            pltpu.SemaphoreType.DMA,
            pltpu.SemaphoreType.DMA,
        ],
        compiler_params=_sc_compiler_params(),
    )
    def k(a_hbm, b_hbm, x_hbm, i0_hbm, i1_hbm, z_hbm, rd_hbm,
          ga0, gb0, ga1, gb1, xt, sga, sgb):
        pltpu.sync_copy(x_hbm, xt)

        def body(i0_v, i1_v, z_v, rd_v):
            def gissue(hs, ga, gb):
                pltpu.async_copy(
                    a_hbm.at[i0_v.at[0, pl.ds(hs, GW // 2)]], ga, sga)
                pltpu.async_copy(
                    b_hbm.at[i1_v.at[0, pl.ds(hs, GW // 2)]], gb, sgb)

            def gdrain(hs, ga, gb):
                pltpu.make_async_copy(
                    a_hbm.at[i0_v.at[0, pl.ds(hs, GW // 2)]], ga, sga).wait()
                pltpu.make_async_copy(
                    b_hbm.at[i1_v.at[0, pl.ds(hs, GW // 2)]], gb, sgb).wait()

            def rdcomp(hs):
                @pl.loop(0, GW // 32)
                def _(g):
                    e0 = hs + g * 16
                    i0 = i0_v[0, pl.ds(e0, 16)]
                    i1 = i1_v[0, pl.ds(e0, 16)]
                    rows = lax.iota(jnp.int32, 16) + e0
                    d = []
                    for c in range(3):
                        f0 = i0 * 3 + c
                        f1 = i1 * 3 + c
                        v0 = plsc.load_gather(xt, [f0 >> 7, f0 & 127])
                        v1 = plsc.load_gather(xt, [f1 >> 7, f1 & 127])
                        d.append(v0 - v1)
                    qsq = d[0] * d[0] + d[1] * d[1] + d[2] * d[2]
                    i32 = jnp.int32(0x5f3759df) - (plsc.bitcast(qsq, jnp.int32) >> 1)
                    y = plsc.bitcast(i32, jnp.float32)
                    for _ in range(3):
                        y = y * (1.5 - 0.5 * qsq * y * y)
                    for c in range(3):
                        dn = d[c] * y
                        plsc.store_scatter(
                            rd_v, [rows, jnp.full((16,), c, jnp.int32)], dn)
                    plsc.store_scatter(
                        rd_v, [rows, jnp.full((16,), 3, jnp.int32)], qsq)

            def zadd(hs, ga, gb):
                @pl.loop(0, GW // 2)
                def _(r):
                    for c in range(DIM // 16):
                        sl = pl.ds(c * 16, 16)
                        z_v[hs + r, sl] = ga[r, sl] + gb[r, sl]

            h1 = GW // 2
            gissue(0, ga0, gb0)
            rdcomp(0)
            gdrain(0, ga0, gb0)
            gissue(h1, ga1, gb1)
            rdcomp(h1)
            zadd(0, ga0, gb0)
            gdrain(h1, ga1, gb1)
            zadd(h1, ga1, gb1)

        pltpu.emit_pipeline(
            body,
            grid=(EP // GW,),
            in_specs=[
                pl.BlockSpec((1, GW), lambda i: (0, i)),
                pl.BlockSpec((1, GW), lambda i: (0, i)),
            ],
            out_specs=[
                pl.BlockSpec((GW, DIM), lambda i: (i, 0)),
                pl.BlockSpec((GW, 16), lambda i: (i, 0)),
            ],
            core_axis_name=("core", "subcore"),
            dimension_semantics=(pltpu.PARALLEL,),
        )(i0_hbm, i1_hbm, z_hbm, rd_hbm)

    return k(a, b, xflat, idx0, idx1)


def _edge_body(z_ref, rd_ref, w0_ref, idx_ref, p0_ref, p1_ref, p2_ref,
               ie_ref):
    z = z_ref[...]
    rd = rd_ref[...]
    d = rd[:, 0:3]
    qsq = rd[:, 3:4]
    zz = z + qsq * w0_ref[...]
    h = _silu(zz)
    p0_ref[...] = h * d[:, 0:1]
    p1_ref[...] = h * d[:, 1:2]
    p2_ref[...] = h * d[:, 2:3]
    idx = idx_ref[...]
    in0 = (idx >= 0) & (idx < NHALF)
    in1 = (idx >= NHALF) & (idx < N)
    ie_ref[0] = jnp.where(in0, idx, TRASH)
    ie_ref[1] = jnp.where(in1, idx - NHALF, TRASH)


def _stage_edge(z, rd, w0, idx0s):
    ieb = pl.BlockSpec((2, BE // 128, 128), lambda i: (0, i, 0))
    ies = jax.ShapeDtypeStruct((2, EP // 128, 128), jnp.int32)
    return pl.pallas_call(
        _edge_body,
        grid=(EP // BE,),
        in_specs=[
            pl.BlockSpec((BE, DIM), lambda i: (i, 0)),
            pl.BlockSpec((BE, 16), lambda i: (i, 0)),
            pl.BlockSpec((1, DIM), lambda i: (0, 0)),
            pl.BlockSpec((BE // 128, 128), lambda i: (i, 0)),
        ],
        out_specs=[
            pl.BlockSpec((BE, DIM), lambda i: (i, 0)),
            pl.BlockSpec((BE, DIM), lambda i: (i, 0)),
            pl.BlockSpec((BE, DIM), lambda i: (i, 0)),
            ieb,
        ],
        out_shape=[
            jax.ShapeDtypeStruct((EP, DIM), jnp.float32),
            jax.ShapeDtypeStruct((EP, DIM), jnp.float32),
            jax.ShapeDtypeStruct((EP, DIM), jnp.float32),
            ies,
        ],
    )(z, rd, w0, idx0s)


def _stage_scatter(p0, p1, p2, rd, ie):
    mesh = plsc.VectorSubcoreMesh(core_axis_name="core",
                                  subcore_axis_name="subcore")
    ck = CH
    n_chunks = EP // (16 * ck)

    @functools.partial(
        pl.kernel,
        out_type=[
            jax.ShapeDtypeStruct((N2, DIM), jnp.float32),
            jax.ShapeDtypeStruct((N2, DIM), jnp.float32),
            jax.ShapeDtypeStruct((N2, DIM), jnp.float32),
            jax.ShapeDtypeStruct((2, 64, ACC_R), jnp.float32),
        ],
        mesh=mesh,
        scratch_types=[
            pltpu.VMEM((CH, DIM), jnp.float32),
            pltpu.VMEM((CH, DIM), jnp.float32),
            pltpu.VMEM((CH, 16), jnp.float32),
            pltpu.VMEM((1, CH), jnp.int32),
            pltpu.VMEM((1, CH), jnp.int32),
            pltpu.VMEM((4, ACC_R), jnp.float32),
            pltpu.VMEM((16, DIM), jnp.float32),
            pltpu.VMEM((40, DIM), jnp.float32),
            pltpu.VMEM_SHARED((ACC_R, DIM), jnp.float32),
            pltpu.SemaphoreType.DMA,
            pltpu.SemaphoreType.DMA,
        ],
        compiler_params=_sc_compiler_params(),
    )
    def k(p0_h, p1_h, p2_h, rd_h, ie_h, s0_h, s1_h, s2_h, dsp_h,
          pba, pbb, rdb, iba, ibb, priv, zb, db, acc, sema, semb):
        c = lax.axis_index("core")
        s = lax.axis_index("subcore")

        @pl.loop(0, 16)
        def _(r):
            for cc in range(DIM // 16):
                zb[r, pl.ds(cc * 16, 16)] = jnp.zeros((16,), jnp.float32)

        @pl.loop(0, 4)
        def _(r):
            @pl.loop(0, ACC_R // 16)
            def _(j):
                priv[r, pl.ds(j * 16, 16)] = jnp.zeros((16,), jnp.float32)

        for kpass, (p_h, s_h) in enumerate(
                ((p0_h, s0_h), (p1_h, s1_h), (p2_h, s2_h))):

            @pl.loop(0, 20)
            def _(t):
                ro = s * 320 + t * 16
                pltpu.sync_copy(zb, acc.at[pl.ds(ro, 16)])

            plsc.subcore_barrier()

            def issue(t, pb_, ib_, sem):
                base = (s * n_chunks + t) * ck
                pltpu.async_copy(ie_h.at[c, pl.ds(base, CH)], ib_.at[0], sem)
                pltpu.async_copy(p_h.at[pl.ds(base, ck)], pb_, sem)

            def drain(t, pb_, ib_, sem):
                base = (s * n_chunks + t) * ck
                pltpu.make_async_copy(ie_h.at[c, pl.ds(base, CH)], ib_.at[0],
                                      sem).wait()
                pltpu.make_async_copy(p_h.at[pl.ds(base, ck)], pb_, sem).wait()

            def scat(pb_, ib_):
                pltpu.sync_copy(pb_, acc.at[ib_.at[0]], add=True)

            def accum_d(t, ib_):
                if kpass != 0:
                    return
                base = (s * n_chunks + t) * ck
                pltpu.sync_copy(rd_h.at[pl.ds(base, ck)], rdb)

                @pl.loop(0, ck // 16)
                def _(g):
                    iloc = ib_[0, pl.ds(g * 16, 16)]
                    rows = lax.iota(jnp.int32, 16) + g * 16
                    for cc in range(3):
                        dc = plsc.load_gather(
                            rdb, [rows, jnp.full((16,), cc, jnp.int32)])
                        plsc.addupdate_scatter(
                            priv, [jnp.full((16,), cc, jnp.int32), iloc], dc)
                    plsc.addupdate_scatter(
                        priv, [jnp.full((16,), 3, jnp.int32), iloc],
                        jnp.full((16,), 1.0, jnp.float32))

            issue(0, pba, iba, sema)

            @pl.loop(0, n_chunks, step=2)
            def _(t):
                issue(t + 1, pbb, ibb, semb)
                drain(t, pba, iba, sema)
                scat(pba, iba)
                accum_d(t, iba)

                @pl.when(t + 2 < n_chunks)
                def _():
                    issue(t + 2, pba, iba, sema)

                drain(t + 1, pbb, ibb, semb)
                scat(pbb, ibb)
                accum_d(t + 1, ibb)

            plsc.subcore_barrier()

            @pl.loop(0, 8)
            def _(t):
                cidx = s + t * 16

                @pl.when(cidx < NHALF // 40)
                def _():
                    ro = cidx * 40
                    pltpu.sync_copy(acc.at[pl.ds(ro, 40)], db)
                    pltpu.sync_copy(db, s_h.at[pl.ds(c * NHALF + ro, 40)])

            plsc.subcore_barrier()

        pltpu.sync_copy(priv, dsp_h.at[c, pl.ds(s * 4, 4)])

    return k(p0, p1, p2, rd, ie.reshape(2, EP))


def _final_body(xh_ref, s0_ref, s1_ref, s2_ref, dsp_ref, sel_ref, w2_ref,
                b2_ref,
                n1w1_ref, n1b1_ref, n1w2_ref, n1b2_ref,
                n2w1_ref, n2b1_ref, n2w2_ref, n2b2_ref,
                n3w1a_ref, n3w1b_ref, n3b1_ref, n3w2_ref, n3b2_ref,
                xo_ref, nh_ref, nrm_ref):
    xh = xh_ref[...]
    sd = lax.dot_general(dsp_ref[...], sel_ref[...],
                         (((0,), (0,)), ((), ())),
                         preferred_element_type=jnp.float32)
    deg = sd[:, 3:4]
    inv = 1.0 / jnp.maximum(deg, 1.0)
    w2 = w2_ref[...]
    b2 = b2_ref[...]
    means = []
    for k, s_ref in enumerate((s0_ref, s1_ref, s2_ref)):
        sk = s_ref[...]
        mk = (jnp.dot(sk, w2, preferred_element_type=jnp.float32)
              + b2 * sd[:, k:k+1]) * inv
        means.append(mk)
    nrm = jnp.sqrt(means[0] * means[0] + means[1] * means[1]
                   + means[2] * means[2])
    nrm_ref[...] = nrm
    t1 = _silu(jnp.dot(xh, n1w1_ref[...], preferred_element_type=jnp.float32)
               + n1b1_ref[...])
    o1 = jnp.dot(t1, n1w2_ref[...], preferred_element_type=jnp.float32) + n1b2_ref[...]
    cols = []
    for k in range(3):
        u = _silu(jnp.dot(means[k], n2w1_ref[...],
                          preferred_element_type=jnp.float32) + n2b1_ref[...])
        cols.append(jnp.dot(u, n2w2_ref[...],
                            preferred_element_type=jnp.float32) + n2b2_ref[...])
    xo_ref[...] = o1 + jnp.concatenate(cols, axis=1)
    g = _silu(jnp.dot(xh, n3w1a_ref[...], preferred_element_type=jnp.float32)
              + jnp.dot(nrm, n3w1b_ref[...], preferred_element_type=jnp.float32)
              + n3b1_ref[...])
    nh_ref[...] = (jnp.dot(g, n3w2_ref[...], preferred_element_type=jnp.float32)
                   + n3b2_ref[...])


def _stage_final(xh2, s0, s1, s2, dsp, sel, w2, b2, n1w1, n1b1, n1w2, n1b2,
                 n2w1, n2b1, n2w2, n2b2, n3w1a, n3w1b, n3b1, n3w2, n3b2):
    full = lambda r, c: pl.BlockSpec((r, c), lambda i: (0, 0))
    blk = lambda c: pl.BlockSpec((BN, c), lambda i: (i, 0))
    return pl.pallas_call(
        _final_body,
        grid=(N2 // BN,),
        in_specs=[
            blk(DIM), blk(DIM), blk(DIM), blk(DIM),
            pl.BlockSpec((64, BN), lambda i: (0, i)),
            full(64, 4),
            full(DIM, DIM), full(1, DIM),
            full(DIM, DIM), full(1, DIM), full(DIM, 3), full(1, 3),
            full(DIM, DIM), full(1, DIM), full(DIM, 1), full(1, 1),
            full(DIM, DIM), full(DIM, DIM), full(1, DIM), full(DIM, DIM),
            full(1, DIM),
        ],
        out_specs=[blk(3), blk(DIM), blk(DIM)],
        out_shape=[
            jax.ShapeDtypeStruct((N2, 3), jnp.float32),
            jax.ShapeDtypeStruct((N2, DIM), jnp.float32),
            jax.ShapeDtypeStruct((N2, DIM), jnp.float32),
        ],
    )(xh2, s0, s1, s2, dsp, sel, w2, b2, n1w1, n1b1, n1w2, n1b2,
      n2w1, n2b1, n2w2, n2b2, n3w1a, n3w1b, n3b1, n3w2, n3b2)


def kernel(x, xh, e, sc_W1, sc_b1, sc_W2, sc_b2, n1_W1, n1_b1, n1_W2, n1_b2,
           n2_W1, n2_b1, n2_W2, n2_b2, n3_W1, n3_b1, n3_W2, n3_b2):
    x2 = x[0]
    xh2 = xh[0]
    w0 = sc_W1[0:1]
    w1a = sc_W1[1:1+DIM]
    w1b = sc_W1[1+DIM:]
    b1 = sc_b1.reshape(1, DIM)

    xh2p = jnp.pad(xh2, ((0, N2 - N), (0, 0)))
    a, b = _stage_ab(xh2p, w1a, w1b, b1)

    xflat = jnp.pad(x2.reshape(3 * N), (0, XT_R * 128 - 3 * N)).reshape(XT_R, 128)
    pad = EP - E
    idx0 = e[0]
    idx1 = e[1]
    idx0g = jnp.pad(idx0, (0, pad), constant_values=N).reshape(1, EP)
    idx1g = jnp.pad(idx1, (0, pad), constant_values=N).reshape(1, EP)
    idx0s = jnp.pad(idx0, (0, pad), constant_values=-1).reshape(EP // 128, 128)
    sel = jnp.tile(jnp.eye(4, dtype=jnp.float32), (16, 1))

    z, rd = _stage_gather(a, b, xflat, idx0g, idx1g)
    p0, p1, p2, ie = _stage_edge(z, rd, w0, idx0s)
    s0, s1, s2, dsp = _stage_scatter(p0, p1, p2, rd, ie)
    dspp = jnp.pad(
        jnp.concatenate([dsp[0, :, :NHALF], dsp[1, :, :NHALF]], axis=1),
        ((0, 0), (0, N2 - N)))

    xo, nh, nrm = _stage_final(
        xh2p, s0, s1, s2, dspp, sel, sc_W2, sc_b2.reshape(1, DIM),
        n1_W1, n1_b1.reshape(1, DIM), n1_W2, n1_b2.reshape(1, 3),
        n2_W1, n2_b1.reshape(1, DIM), n2_W2, n2_b2.reshape(1, 1),
        n3_W1[:DIM], n3_W1[DIM:], n3_b1.reshape(1, DIM),
        n3_W2, n3_b2.reshape(1, DIM))

    return (xo[:N][None], nh[:N][None], nrm[:N][None])
